# Initial kernel scaffold; baseline (speedup 1.0000x reference)
import jax
import jax.numpy as jnp
from jax.experimental import pallas as pl


def kernel(x, edge_index, edge_attr, batch, recipe, lengths, stats, baseline, W1, b1, W2, b2, We, be, V1, c1, V2, c2, emb, Wi, Wh, bL, S1, sb1, S2, sb2, H1, hb1, H2, hb2, H3, hb3):
    raise NotImplementedError("write your pallas kernel here")


# speedup vs baseline: 1.4177x; 1.4177x over previous
import functools

import jax
import jax.numpy as jnp
from jax import lax
from jax.experimental import pallas as pl
from jax.experimental.pallas import tpu as pltpu
from jax.experimental.pallas import tpu_sc as plsc

N = 100000
E = 3200000
EBLK = 128
EP = 782 * 32 * EBLK
NACC = 100008
B = 64
L = 50
VOCABP = 1024
NBLK = 1000
NGRID = N // NBLK

_SC_MESH = dict(core_axis_name="c", subcore_axis_name="s")


def _sc_conv1(src_p, dst_p, attr_flat, xpad, zeros_acc):
    mesh = plsc.VectorSubcoreMesh(**_SC_MESH)
    per_tile = EP // 32
    nblk = per_tile // EBLK

    @functools.partial(
        pl.kernel,
        out_type=(jax.ShapeDtypeStruct((N, 16), jnp.float32),
                  jax.ShapeDtypeStruct((N, 16), jnp.float32)),
        mesh=mesh,
        scratch_types=[
            pltpu.VMEM((EBLK,), jnp.int32),
            pltpu.VMEM((EBLK,), jnp.int32),
            pltpu.VMEM((EBLK * 5 + 16,), jnp.float32),
            pltpu.VMEM((EBLK, 16), jnp.float32),
            pltpu.VMEM((EBLK, 16), jnp.float32),
            pltpu.VMEM_SHARED((NACC, 16), jnp.float32),
            pltpu.SemaphoreType.DMA,
        ],
        compiler_params=pltpu.CompilerParams(use_tc_tiling_on_sc=False),
    )
    def k(src_h, dst_h, attr_h, x_h, z_h, out0, out1,
          srcb, dstb, attrb, rows, msg, acc, sem):
        c = lax.axis_index("c")
        s = lax.axis_index("s")

        @pl.when(s == 0)
        def _():
            pltpu.sync_copy(z_h, acc)
        plsc.subcore_barrier()

        base = (c * 16 + s) * per_tile
        fmask = lax.broadcasted_iota(jnp.int32, (16,), 0) < 5

        def blk(i, carry):
            e0 = base + i * EBLK
            pltpu.sync_copy(src_h.at[pl.ds(e0, EBLK)], srcb)
            pltpu.sync_copy(dst_h.at[pl.ds(e0, EBLK)], dstb)
            pltpu.sync_copy(attr_h.at[pl.ds(e0 * 5, EBLK * 5)],
                            attrb.at[pl.ds(0, EBLK * 5)])
            pltpu.async_copy(x_h.at[srcb], rows, sem).wait()

            def inner(e, carry2):
                a = attrb[pl.ds(e * 5, 16)]
                a = jnp.where(fmask, a, 0.0)
                msg[e] = jnp.maximum(rows[e] + a, 0.0)
                return carry2

            lax.fori_loop(0, EBLK, inner, 0, unroll=4)
            pltpu.sync_copy(msg, acc.at[dstb], add=True)
            return carry

        lax.fori_loop(0, nblk, blk, 0)
        plsc.subcore_barrier()

        @pl.when(jnp.logical_and(s == 0, c == 0))
        def _():
            pltpu.sync_copy(acc.at[pl.ds(0, N)], out0)

        @pl.when(jnp.logical_and(s == 0, c == 1))
        def _():
            pltpu.sync_copy(acc.at[pl.ds(0, N)], out1)

    return k(src_p, dst_p, attr_flat, xpad, zeros_acc)


def _sc_conv2(src_p, dst_p, attr_flat, h_cm, We_cm, be_cm, zeros_acc):
    mesh = plsc.VectorSubcoreMesh(**_SC_MESH)
    per_tile = EP // 16
    nblk = per_tile // EBLK

    @functools.partial(
        pl.kernel,
        out_type=jax.ShapeDtypeStruct((8, N, 16), jnp.float32),
        mesh=mesh,
        scratch_types=[
            pltpu.VMEM((EBLK,), jnp.int32),
            pltpu.VMEM((EBLK,), jnp.int32),
            pltpu.VMEM((EBLK,), jnp.int32),
            pltpu.VMEM((EBLK * 5 + 16,), jnp.float32),
            pltpu.VMEM((EBLK, 16), jnp.float32),
            pltpu.VMEM((EBLK, 16), jnp.float32),
            pltpu.VMEM((5, 16), jnp.float32),
            pltpu.VMEM((16,), jnp.float32),
            pltpu.VMEM_SHARED((NACC, 16), jnp.float32),
            pltpu.SemaphoreType.DMA,
        ],
        compiler_params=pltpu.CompilerParams(use_tc_tiling_on_sc=False),
    )
    def k(src_h, dst_h, attr_h, hcm_h, we_h, be_h, z_h, out,
          srcb, dstb, idxb, attrb, rows, msg, wev, bev, acc, sem):
        c = lax.axis_index("c")
        s = lax.axis_index("s")
        base = s * per_tile

        for p in range(4):
            @pl.when(s == 0)
            def _():
                pltpu.sync_copy(z_h, acc)

            @pl.when(jnp.logical_and(s == 0, c == 0))
            def _():
                pltpu.sync_copy(we_h.at[2 * p], wev)
                pltpu.sync_copy(be_h.at[2 * p], bev)

            @pl.when(jnp.logical_and(s == 0, c == 1))
            def _():
                pltpu.sync_copy(we_h.at[2 * p + 1], wev)
                pltpu.sync_copy(be_h.at[2 * p + 1], bev)
            plsc.subcore_barrier()

            @pl.when(jnp.logical_and(s != 0, c == 0))
            def _():
                pltpu.sync_copy(we_h.at[2 * p], wev)
                pltpu.sync_copy(be_h.at[2 * p], bev)

            @pl.when(jnp.logical_and(s != 0, c == 1))
            def _():
                pltpu.sync_copy(we_h.at[2 * p + 1], wev)
                pltpu.sync_copy(be_h.at[2 * p + 1], bev)

            off = (2 * p + c) * N
            we0 = wev[0]
            we1 = wev[1]
            we2 = wev[2]
            we3 = wev[3]
            we4 = wev[4]
            bevv = bev[pl.ds(0, 16)]

            def blk(i, carry):
                e0 = base + i * EBLK
                pltpu.sync_copy(src_h.at[pl.ds(e0, EBLK)], srcb)
                pltpu.sync_copy(dst_h.at[pl.ds(e0, EBLK)], dstb)
                pltpu.sync_copy(attr_h.at[pl.ds(e0 * 5, EBLK * 5)],
                                attrb.at[pl.ds(0, EBLK * 5)])

                def adj(j, carry2):
                    idxb[pl.ds(j * 16, 16)] = srcb[pl.ds(j * 16, 16)] + off
                    return carry2

                lax.fori_loop(0, EBLK // 16, adj, 0, unroll=True)
                pltpu.async_copy(hcm_h.at[idxb], rows, sem).wait()

                def inner(e, carry2):
                    a = attrb[pl.ds(e * 5, 16)]
                    v = rows[e] + bevv + a[0] * we0 + a[1] * we1 \
                        + a[2] * we2 + a[3] * we3 + a[4] * we4
                    msg[e] = jnp.maximum(v, 0.0)
                    return carry2

                lax.fori_loop(0, EBLK, inner, 0, unroll=4)
                pltpu.sync_copy(msg, acc.at[dstb], add=True)
                return carry

            lax.fori_loop(0, nblk, blk, 0)
            plsc.subcore_barrier()

            @pl.when(jnp.logical_and(s == 0, c == 0))
            def _():
                pltpu.sync_copy(acc.at[pl.ds(0, N)], out.at[2 * p])

            @pl.when(jnp.logical_and(s == 0, c == 1))
            def _():
                pltpu.sync_copy(acc.at[pl.ds(0, N)], out.at[2 * p + 1])
            plsc.subcore_barrier()

    return k(src_p, dst_p, attr_flat, h_cm, We_cm, be_cm, zeros_acc)


def _mlp1(xpad, a0, a1, W1p, b1r, W2, b2r):
    def body(x_ref, a0_ref, a1_ref, w1_ref, b1_ref, w2_ref, b2_ref, o_ref):
        X = x_ref[...] + a0_ref[...] + a1_ref[...]
        t = jnp.maximum(jnp.dot(X, w1_ref[...],
                                preferred_element_type=jnp.float32)
                        + b1_ref[...], 0.0)
        o_ref[...] = jnp.maximum(jnp.dot(t, w2_ref[...],
                                         preferred_element_type=jnp.float32)
                                 + b2_ref[...], 0.0)

    return pl.pallas_call(
        body,
        grid=(NGRID,),
        in_specs=[
            pl.BlockSpec((NBLK, 16), lambda i: (i, 0)),
            pl.BlockSpec((NBLK, 16), lambda i: (i, 0)),
            pl.BlockSpec((NBLK, 16), lambda i: (i, 0)),
            pl.BlockSpec((16, 128), lambda i: (0, 0)),
            pl.BlockSpec((1, 128), lambda i: (0, 0)),
            pl.BlockSpec((128, 128), lambda i: (0, 0)),
            pl.BlockSpec((1, 128), lambda i: (0, 0)),
        ],
        out_specs=pl.BlockSpec((NBLK, 128), lambda i: (i, 0)),
        out_shape=jax.ShapeDtypeStruct((N, 128), jnp.float32),
    )(xpad, a0, a1, W1p, b1r, W2, b2r)


def _mlp2pool(h, agg2, batch3, V1, c1r, V2, c2r):
    def body(h_ref, a_ref, b_ref, v1_ref, c1_ref, v2_ref, c2_ref,
             sums_ref, cnts_ref):
        @pl.when(pl.program_id(0) == 0)
        def _():
            sums_ref[...] = jnp.zeros_like(sums_ref)
            cnts_ref[...] = jnp.zeros_like(cnts_ref)

        X = h_ref[...] + a_ref[...]
        t = jnp.maximum(jnp.dot(X, v1_ref[...],
                                preferred_element_type=jnp.float32)
                        + c1_ref[...], 0.0)
        h2 = jnp.maximum(jnp.dot(t, v2_ref[...],
                                 preferred_element_type=jnp.float32)
                         + c2_ref[...], 0.0)
        bb = b_ref[...].reshape(1, NBLK)
        ohT = (jnp.broadcast_to(bb, (B, NBLK))
               == lax.broadcasted_iota(jnp.int32, (B, NBLK), 0)
               ).astype(jnp.float32)
        sums_ref[...] += jnp.dot(ohT, h2, preferred_element_type=jnp.float32)
        cnt = jnp.sum(ohT, axis=1, keepdims=True)
        cnts_ref[...] += jnp.broadcast_to(cnt, (B, 128))

    return pl.pallas_call(
        body,
        grid=(NGRID,),
        in_specs=[
            pl.BlockSpec((NBLK, 128), lambda i: (i, 0)),
            pl.BlockSpec((NBLK, 128), lambda i: (i, 0)),
            pl.BlockSpec((1, 1, NBLK), lambda i: (i, 0, 0)),
            pl.BlockSpec((128, 128), lambda i: (0, 0)),
            pl.BlockSpec((1, 128), lambda i: (0, 0)),
            pl.BlockSpec((128, 128), lambda i: (0, 0)),
            pl.BlockSpec((1, 128), lambda i: (0, 0)),
        ],
        out_specs=[
            pl.BlockSpec((B, 128), lambda i: (0, 0)),
            pl.BlockSpec((B, 128), lambda i: (0, 0)),
        ],
        out_shape=[
            jax.ShapeDtypeStruct((B, 128), jnp.float32),
            jax.ShapeDtypeStruct((B, 128), jnp.float32),
        ],
        compiler_params=pltpu.CompilerParams(
            dimension_semantics=("arbitrary",)),
    )(h, agg2, batch3, V1, c1r, V2, c2r)


def _final(sums, cnts, recipe, len_col, stats_p, baseline, emb_pad,
           Wi, Wh, bLr, S1p, sb1r, S2, sb2r,
           H1g, H1r, H1s, H1b, hb1r, H2, hb2r, H3p, hb3r):
    def body(sums_ref, cnts_ref, rec_ref, len_ref, st_ref, base_ref, emb_ref,
             wi_ref, wh_ref, bl_ref, s1_ref, sb1_ref, s2_ref, sb2_ref,
             h1g_ref, h1r_ref, h1s_ref, h1b_ref, hb1_ref, h2_ref, hb2_ref,
             h3_ref, hb3_ref, o_ref):
        g = sums_ref[...] / jnp.maximum(cnts_ref[...], 1.0)

        idxc = jnp.clip(len_ref[...] - 1, 0, L - 1)
        wi = wi_ref[...]
        wh = wh_ref[...]
        bl = bl_ref[...]
        emb = emb_ref[...]

        def sigmoid(v):
            return 1.0 / (1.0 + jnp.exp(-v))

        rec = rec_ref[...]
        zero = jnp.zeros((B, 64), jnp.float32)
        hh, cc, sel = zero, zero, zero
        for t in range(L):
            rt = rec[:, t:t + 1]
            oh = (jnp.broadcast_to(rt, (B, VOCABP))
                  == lax.broadcasted_iota(jnp.int32, (B, VOCABP), 1)
                  ).astype(jnp.float32)
            xt = jnp.dot(oh, emb, preferred_element_type=jnp.float32)
            z = (jnp.dot(xt, wi, preferred_element_type=jnp.float32)
                 + jnp.dot(hh, wh, preferred_element_type=jnp.float32) + bl)
            i_ = sigmoid(z[:, 0:64])
            f_ = sigmoid(z[:, 64:128])
            g_ = jnp.tanh(z[:, 128:192])
            o_ = sigmoid(z[:, 192:256])
            cc = f_ * cc + i_ * g_
            hh = o_ * jnp.tanh(cc)
            sel = jnp.where(idxc == t, hh, sel)
        r = sel

        st = jnp.maximum(jnp.dot(st_ref[...], s1_ref[...],
                                 preferred_element_type=jnp.float32)
                         + sb1_ref[...], 0.0)
        s = jnp.dot(st, s2_ref[...], preferred_element_type=jnp.float32) \
            + sb2_ref[...]

        z1 = (jnp.dot(g, h1g_ref[...], preferred_element_type=jnp.float32)
              + jnp.dot(r, h1r_ref[...], preferred_element_type=jnp.float32)
              + jnp.dot(s, h1s_ref[...], preferred_element_type=jnp.float32)
              + base_ref[...] * h1b_ref[...]
              + hb1_ref[...])
        o1 = jnp.maximum(z1, 0.0)
        o2 = jnp.maximum(jnp.dot(o1, h2_ref[...],
                                 preferred_element_type=jnp.float32)
                         + hb2_ref[...], 0.0)
        o_ref[...] = jnp.dot(o2, h3_ref[...],
                             preferred_element_type=jnp.float32) + hb3_ref[...]

    return pl.pallas_call(
        body,
        out_shape=jax.ShapeDtypeStruct((B, 128), jnp.float32),
    )(sums, cnts, recipe, len_col, stats_p, baseline, emb_pad,
      Wi, Wh, bLr, S1p, sb1r, S2, sb2r,
      H1g, H1r, H1s, H1b, hb1r, H2, hb2r, H3p, hb3r)


def kernel(x, edge_index, edge_attr, batch, recipe, lengths, stats, baseline,
           W1, b1, W2, b2, We, be, V1, c1, V2, c2, emb, Wi, Wh, bL,
           S1, sb1, S2, sb2, H1, hb1, H2, hb2, H3, hb3):
    pad = EP - E
    src_p = jnp.concatenate([edge_index[0], jnp.zeros((pad,), jnp.int32)])
    dst_p = jnp.concatenate([edge_index[1],
                             jnp.full((pad,), N, jnp.int32)])
    attr_flat = jnp.concatenate([edge_attr.reshape(-1),
                                 jnp.zeros((pad * 5 + 16,), jnp.float32)])
    xpad = jnp.pad(x, ((0, 0), (0, 11)))
    zeros_acc = jnp.zeros((NACC, 16), jnp.float32)

    a0, a1 = _sc_conv1(src_p, dst_p, attr_flat, xpad, zeros_acc)

    W1p = jnp.pad(W1, ((0, 11), (0, 0)))
    h = _mlp1(xpad, a0, a1, W1p, b1.reshape(1, 128), W2, b2.reshape(1, 128))

    h_cm = h.reshape(N, 8, 16).transpose(1, 0, 2).reshape(8 * N, 16)
    We_cm = We.reshape(5, 8, 16).transpose(1, 0, 2)
    be_cm = be.reshape(8, 16)
    agg2_cm = _sc_conv2(src_p, dst_p, attr_flat, h_cm, We_cm, be_cm,
                        zeros_acc)
    agg2 = agg2_cm.transpose(1, 0, 2).reshape(N, 128)

    batch3 = batch.reshape(NGRID, 1, NBLK)
    sums, cnts = _mlp2pool(h, agg2, batch3, V1, c1.reshape(1, 128),
                           V2, c2.reshape(1, 128))

    emb_pad = jnp.pad(emb, ((0, VOCABP - (emb.shape[0])), (0, 0)))
    stats_p = jnp.pad(stats, ((0, 0), (0, 2)))
    S1p = jnp.pad(S1, ((0, 2), (0, 0)))
    H1g = H1[0:128]
    H1r = H1[128:192]
    H1s = H1[192:224]
    H1b = H1[224:225]
    H3p = jnp.pad(H3, ((0, 0), (0, 127)))
    hb3r = jnp.pad(hb3, (0, 127)).reshape(1, 128)

    out128 = _final(sums, cnts, recipe, lengths.reshape(B, 1),
                    stats_p, baseline, emb_pad,
                    Wi, Wh, bL.reshape(1, 256),
                    S1p, sb1.reshape(1, 32), S2, sb2.reshape(1, 32),
                    H1g, H1r, H1s, H1b, hb1.reshape(1, 128),
                    H2, hb2.reshape(1, 64), H3p, hb3r)
    return out128[:, 0:1]


# speedup vs baseline: 2.4094x; 1.6995x over previous
import functools

import jax
import jax.numpy as jnp
from jax import lax
from jax.experimental import pallas as pl
from jax.experimental.pallas import tpu as pltpu
from jax.experimental.pallas import tpu_sc as plsc

N = 100000
E = 3200000
EBLK = 128
EP = 782 * 32 * EBLK
NACC = 100008
B = 64
L = 50
VOCABP = 1024
NBLK = 1000
NGRID = N // NBLK

_SC_MESH = dict(core_axis_name="c", subcore_axis_name="s")


NB5 = EBLK * 5


def _sc_conv1(src_p, dst_p, attr_flat, xpad, zeros_acc):
    mesh = plsc.VectorSubcoreMesh(**_SC_MESH)
    per_tile = EP // 32
    nblk = per_tile // EBLK

    @functools.partial(
        pl.kernel,
        out_type=(jax.ShapeDtypeStruct((N, 16), jnp.float32),
                  jax.ShapeDtypeStruct((N, 16), jnp.float32)),
        mesh=mesh,
        scratch_types=[
            pltpu.VMEM((EBLK,), jnp.int32),
            pltpu.VMEM((EBLK,), jnp.int32),
            pltpu.VMEM((EBLK,), jnp.int32),
            pltpu.VMEM((EBLK,), jnp.int32),
            pltpu.VMEM((NB5 + 16,), jnp.float32),
            pltpu.VMEM((NB5 + 16,), jnp.float32),
            pltpu.VMEM((EBLK, 16), jnp.float32),
            pltpu.VMEM((EBLK, 16), jnp.float32),
            pltpu.VMEM((EBLK, 16), jnp.float32),
            pltpu.VMEM((EBLK, 16), jnp.float32),
            pltpu.VMEM_SHARED((NACC, 16), jnp.float32),
            pltpu.SemaphoreType.DMA,
            pltpu.SemaphoreType.DMA,
            pltpu.SemaphoreType.DMA,
            pltpu.SemaphoreType.DMA,
        ],
        compiler_params=pltpu.CompilerParams(use_tc_tiling_on_sc=False, needs_layout_passes=False),
    )
    def k(src_h, dst_h, attr_h, x_h, z_h, out0, out1,
          sb0, sb1, db0, db1, ab0, ab1, rb0, rb1, mb0, mb1, acc,
          sl0, sl1, sg0, sg1):
        c = lax.axis_index("c")
        s = lax.axis_index("s")
        S = [sb0, sb1]
        D = [db0, db1]
        A = [ab0, ab1]
        R = [rb0, rb1]
        M = [mb0, mb1]
        SL = [sl0, sl1]
        SG = [sg0, sg1]

        @pl.when(s == 0)
        def _():
            pltpu.sync_copy(z_h, acc)
        plsc.subcore_barrier()

        base = (c * 16 + s) * per_tile
        fmask = lax.broadcasted_iota(jnp.int32, (16,), 0) < 5

        def lin_start(i, b):
            e0 = base + jnp.minimum(i, nblk - 1) * EBLK
            pltpu.async_copy(src_h.at[pl.ds(e0, EBLK)], S[b], SL[b])
            pltpu.async_copy(dst_h.at[pl.ds(e0, EBLK)], D[b], SL[b])
            pltpu.async_copy(attr_h.at[pl.ds(e0 * 5, NB5)],
                             A[b].at[pl.ds(0, NB5)], SL[b])

        def lin_wait(b):
            pltpu.make_async_copy(src_h.at[pl.ds(0, EBLK)], S[b],
                                  SL[b]).wait()
            pltpu.make_async_copy(dst_h.at[pl.ds(0, EBLK)], D[b],
                                  SL[b]).wait()
            pltpu.make_async_copy(attr_h.at[pl.ds(0, NB5)],
                                  A[b].at[pl.ds(0, NB5)], SL[b]).wait()

        def g_start(b):
            pltpu.async_copy(x_h.at[S[b]], R[b], SG[b])

        def g_wait(b):
            pltpu.make_async_copy(x_h.at[S[b]], R[b], SG[b]).wait()

        def compute(b):
            rows = R[b]
            msg = M[b]
            attrb = A[b]

            def inner(e, carry2):
                a = attrb[pl.ds(e * 5, 16)]
                a = jnp.where(fmask, a, 0.0)
                msg[e] = jnp.maximum(rows[e] + a, 0.0)
                return carry2

            lax.fori_loop(0, EBLK, inner, 0, unroll=4)

        lin_start(0, 0)
        lin_wait(0)
        g_start(0)
        lin_start(1, 1)

        def pair(j, carry):
            for b in (0, 1):
                i = 2 * j + b
                lin_wait(1 - b)
                g_start(1 - b)
                g_wait(b)
                compute(b)
                pltpu.sync_copy(M[b], acc.at[D[b]], add=True)
                lin_start(i + 2, b)
            return carry

        lax.fori_loop(0, nblk // 2, pair, 0)
        g_wait(0)
        lin_wait(1)
        plsc.subcore_barrier()

        @pl.when(jnp.logical_and(s == 0, c == 0))
        def _():
            pltpu.sync_copy(acc.at[pl.ds(0, N)], out0)

        @pl.when(jnp.logical_and(s == 0, c == 1))
        def _():
            pltpu.sync_copy(acc.at[pl.ds(0, N)], out1)

    return k(src_p, dst_p, attr_flat, xpad, zeros_acc)


def _sc_conv2(src_p, dst_p, attr_flat, h_cm, We_cm, be_cm, zeros_acc):
    mesh = plsc.VectorSubcoreMesh(**_SC_MESH)
    per_tile = EP // 16
    nblk = per_tile // EBLK

    @functools.partial(
        pl.kernel,
        out_type=jax.ShapeDtypeStruct((8, N, 16), jnp.float32),
        mesh=mesh,
        scratch_types=[
            pltpu.VMEM((EBLK,), jnp.int32),
            pltpu.VMEM((EBLK,), jnp.int32),
            pltpu.VMEM((EBLK,), jnp.int32),
            pltpu.VMEM((EBLK,), jnp.int32),
            pltpu.VMEM((EBLK,), jnp.int32),
            pltpu.VMEM((EBLK,), jnp.int32),
            pltpu.VMEM((NB5 + 16,), jnp.float32),
            pltpu.VMEM((NB5 + 16,), jnp.float32),
            pltpu.VMEM((EBLK, 16), jnp.float32),
            pltpu.VMEM((EBLK, 16), jnp.float32),
            pltpu.VMEM((EBLK, 16), jnp.float32),
            pltpu.VMEM((EBLK, 16), jnp.float32),
            pltpu.VMEM((5, 16), jnp.float32),
            pltpu.VMEM((16,), jnp.float32),
            pltpu.VMEM_SHARED((NACC, 16), jnp.float32),
            pltpu.SemaphoreType.DMA,
            pltpu.SemaphoreType.DMA,
            pltpu.SemaphoreType.DMA,
            pltpu.SemaphoreType.DMA,
        ],
        compiler_params=pltpu.CompilerParams(use_tc_tiling_on_sc=False, needs_layout_passes=False),
    )
    def k(src_h, dst_h, attr_h, hcm_h, we_h, be_h, z_h, out,
          sb0, sb1, db0, db1, ib0, ib1, ab0, ab1, rb0, rb1, mb0, mb1,
          wev, bev, acc, sl0, sl1, sg0, sg1):
        c = lax.axis_index("c")
        s = lax.axis_index("s")
        base = s * per_tile
        S = [sb0, sb1]
        D = [db0, db1]
        I = [ib0, ib1]
        A = [ab0, ab1]
        R = [rb0, rb1]
        M = [mb0, mb1]
        SL = [sl0, sl1]
        SG = [sg0, sg1]

        def lin_start(i, b):
            e0 = base + jnp.minimum(i, nblk - 1) * EBLK
            pltpu.async_copy(src_h.at[pl.ds(e0, EBLK)], S[b], SL[b])
            pltpu.async_copy(dst_h.at[pl.ds(e0, EBLK)], D[b], SL[b])
            pltpu.async_copy(attr_h.at[pl.ds(e0 * 5, NB5)],
                             A[b].at[pl.ds(0, NB5)], SL[b])

        def lin_wait(b):
            pltpu.make_async_copy(src_h.at[pl.ds(0, EBLK)], S[b],
                                  SL[b]).wait()
            pltpu.make_async_copy(dst_h.at[pl.ds(0, EBLK)], D[b],
                                  SL[b]).wait()
            pltpu.make_async_copy(attr_h.at[pl.ds(0, NB5)],
                                  A[b].at[pl.ds(0, NB5)], SL[b]).wait()

        def g_start(b, off):
            def adj(j, carry2):
                I[b][pl.ds(j * 16, 16)] = S[b][pl.ds(j * 16, 16)] + off
                return carry2

            lax.fori_loop(0, EBLK // 16, adj, 0, unroll=True)
            pltpu.async_copy(hcm_h.at[I[b]], R[b], SG[b])

        def g_wait(b):
            pltpu.make_async_copy(hcm_h.at[I[b]], R[b], SG[b]).wait()

        for p in range(4):
            @pl.when(s == 0)
            def _():
                pltpu.sync_copy(z_h, acc)

            @pl.when(c == 0)
            def _():
                pltpu.sync_copy(we_h.at[2 * p], wev)
                pltpu.sync_copy(be_h.at[2 * p], bev)

            @pl.when(c == 1)
            def _():
                pltpu.sync_copy(we_h.at[2 * p + 1], wev)
                pltpu.sync_copy(be_h.at[2 * p + 1], bev)
            plsc.subcore_barrier()

            off = (2 * p + c) * N
            we0 = wev[0]
            we1 = wev[1]
            we2 = wev[2]
            we3 = wev[3]
            we4 = wev[4]
            bevv = bev[pl.ds(0, 16)]
            c5 = jnp.full((16,), 5, jnp.int32)
            iv_init = tuple(
                jnp.full((16,), kk, jnp.int32) for kk in range(5))

            def compute(b):
                rows = R[b]
                msg = M[b]
                attrb = A[b]

                def inner(e, carry2):
                    j0, j1, j2, j3, j4 = carry2
                    a0 = plsc.load_gather(attrb, [j0])
                    a1 = plsc.load_gather(attrb, [j1])
                    a2 = plsc.load_gather(attrb, [j2])
                    a3 = plsc.load_gather(attrb, [j3])
                    a4 = plsc.load_gather(attrb, [j4])
                    v = rows[e] + bevv + a0 * we0 + a1 * we1 \
                        + a2 * we2 + a3 * we3 + a4 * we4
                    msg[e] = jnp.maximum(v, 0.0)
                    return (j0 + c5, j1 + c5, j2 + c5, j3 + c5, j4 + c5)

                lax.fori_loop(0, EBLK, inner, iv_init, unroll=4)

            lin_start(0, 0)
            lin_wait(0)
            g_start(0, off)
            lin_start(1, 1)

            def pair(j, carry):
                for b in (0, 1):
                    i = 2 * j + b
                    lin_wait(1 - b)
                    g_start(1 - b, off)
                    g_wait(b)
                    compute(b)
                    pltpu.sync_copy(M[b], acc.at[D[b]], add=True)
                    lin_start(i + 2, b)
                return carry

            lax.fori_loop(0, nblk // 2, pair, 0)
            g_wait(0)
            lin_wait(1)
            plsc.subcore_barrier()

            @pl.when(jnp.logical_and(s == 0, c == 0))
            def _():
                pltpu.sync_copy(acc.at[pl.ds(0, N)], out.at[2 * p])

            @pl.when(jnp.logical_and(s == 0, c == 1))
            def _():
                pltpu.sync_copy(acc.at[pl.ds(0, N)], out.at[2 * p + 1])
            plsc.subcore_barrier()

    return k(src_p, dst_p, attr_flat, h_cm, We_cm, be_cm, zeros_acc)


def _mlp1(xpad, a0, a1, W1p, b1r, W2, b2r):
    def body(x_ref, a0_ref, a1_ref, w1_ref, b1_ref, w2_ref, b2_ref, o_ref):
        X = x_ref[...] + a0_ref[...] + a1_ref[...]
        t = jnp.maximum(jnp.dot(X, w1_ref[...],
                                preferred_element_type=jnp.float32)
                        + b1_ref[...], 0.0)
        o_ref[...] = jnp.maximum(jnp.dot(t, w2_ref[...],
                                         preferred_element_type=jnp.float32)
                                 + b2_ref[...], 0.0)

    return pl.pallas_call(
        body,
        grid=(NGRID,),
        in_specs=[
            pl.BlockSpec((NBLK, 16), lambda i: (i, 0)),
            pl.BlockSpec((NBLK, 16), lambda i: (i, 0)),
            pl.BlockSpec((NBLK, 16), lambda i: (i, 0)),
            pl.BlockSpec((16, 128), lambda i: (0, 0)),
            pl.BlockSpec((1, 128), lambda i: (0, 0)),
            pl.BlockSpec((128, 128), lambda i: (0, 0)),
            pl.BlockSpec((1, 128), lambda i: (0, 0)),
        ],
        out_specs=pl.BlockSpec((NBLK, 128), lambda i: (i, 0)),
        out_shape=jax.ShapeDtypeStruct((N, 128), jnp.float32),
    )(xpad, a0, a1, W1p, b1r, W2, b2r)


def _mlp2pool(h, agg2, batch3, V1, c1r, V2, c2r):
    def body(h_ref, a_ref, b_ref, v1_ref, c1_ref, v2_ref, c2_ref,
             sums_ref, cnts_ref):
        @pl.when(pl.program_id(0) == 0)
        def _():
            sums_ref[...] = jnp.zeros_like(sums_ref)
            cnts_ref[...] = jnp.zeros_like(cnts_ref)

        X = h_ref[...] + a_ref[...]
        t = jnp.maximum(jnp.dot(X, v1_ref[...],
                                preferred_element_type=jnp.float32)
                        + c1_ref[...], 0.0)
        h2 = jnp.maximum(jnp.dot(t, v2_ref[...],
                                 preferred_element_type=jnp.float32)
                         + c2_ref[...], 0.0)
        bb = b_ref[...].reshape(1, NBLK)
        ohT = (jnp.broadcast_to(bb, (B, NBLK))
               == lax.broadcasted_iota(jnp.int32, (B, NBLK), 0)
               ).astype(jnp.float32)
        sums_ref[...] += jnp.dot(ohT, h2, preferred_element_type=jnp.float32)
        cnt = jnp.sum(ohT, axis=1, keepdims=True)
        cnts_ref[...] += jnp.broadcast_to(cnt, (B, 128))

    return pl.pallas_call(
        body,
        grid=(NGRID,),
        in_specs=[
            pl.BlockSpec((NBLK, 128), lambda i: (i, 0)),
            pl.BlockSpec((NBLK, 128), lambda i: (i, 0)),
            pl.BlockSpec((1, 1, NBLK), lambda i: (i, 0, 0)),
            pl.BlockSpec((128, 128), lambda i: (0, 0)),
            pl.BlockSpec((1, 128), lambda i: (0, 0)),
            pl.BlockSpec((128, 128), lambda i: (0, 0)),
            pl.BlockSpec((1, 128), lambda i: (0, 0)),
        ],
        out_specs=[
            pl.BlockSpec((B, 128), lambda i: (0, 0)),
            pl.BlockSpec((B, 128), lambda i: (0, 0)),
        ],
        out_shape=[
            jax.ShapeDtypeStruct((B, 128), jnp.float32),
            jax.ShapeDtypeStruct((B, 128), jnp.float32),
        ],
        compiler_params=pltpu.CompilerParams(
            dimension_semantics=("arbitrary",)),
    )(h, agg2, batch3, V1, c1r, V2, c2r)


def _final(sums, cnts, recipe, len_col, stats_p, baseline, emb_pad,
           Wi, Wh, bLr, S1p, sb1r, S2, sb2r,
           H1g, H1r, H1s, H1b, hb1r, H2, hb2r, H3p, hb3r):
    def body(sums_ref, cnts_ref, rec_ref, len_ref, st_ref, base_ref, emb_ref,
             wi_ref, wh_ref, bl_ref, s1_ref, sb1_ref, s2_ref, sb2_ref,
             h1g_ref, h1r_ref, h1s_ref, h1b_ref, hb1_ref, h2_ref, hb2_ref,
             h3_ref, hb3_ref, o_ref):
        g = sums_ref[...] / jnp.maximum(cnts_ref[...], 1.0)

        idxc = jnp.clip(len_ref[...] - 1, 0, L - 1)
        wi = wi_ref[...]
        wh = wh_ref[...]
        bl = bl_ref[...]
        emb = emb_ref[...]

        def sigmoid(v):
            return 1.0 / (1.0 + jnp.exp(-v))

        rec = rec_ref[...]
        zero = jnp.zeros((B, 64), jnp.float32)
        hh, cc, sel = zero, zero, zero
        for t in range(L):
            rt = rec[:, t:t + 1]
            oh = (jnp.broadcast_to(rt, (B, VOCABP))
                  == lax.broadcasted_iota(jnp.int32, (B, VOCABP), 1)
                  ).astype(jnp.float32)
            xt = jnp.dot(oh, emb, preferred_element_type=jnp.float32)
            z = (jnp.dot(xt, wi, preferred_element_type=jnp.float32)
                 + jnp.dot(hh, wh, preferred_element_type=jnp.float32) + bl)
            i_ = sigmoid(z[:, 0:64])
            f_ = sigmoid(z[:, 64:128])
            g_ = jnp.tanh(z[:, 128:192])
            o_ = sigmoid(z[:, 192:256])
            cc = f_ * cc + i_ * g_
            hh = o_ * jnp.tanh(cc)
            sel = jnp.where(idxc == t, hh, sel)
        r = sel

        st = jnp.maximum(jnp.dot(st_ref[...], s1_ref[...],
                                 preferred_element_type=jnp.float32)
                         + sb1_ref[...], 0.0)
        s = jnp.dot(st, s2_ref[...], preferred_element_type=jnp.float32) \
            + sb2_ref[...]

        z1 = (jnp.dot(g, h1g_ref[...], preferred_element_type=jnp.float32)
              + jnp.dot(r, h1r_ref[...], preferred_element_type=jnp.float32)
              + jnp.dot(s, h1s_ref[...], preferred_element_type=jnp.float32)
              + base_ref[...] * h1b_ref[...]
              + hb1_ref[...])
        o1 = jnp.maximum(z1, 0.0)
        o2 = jnp.maximum(jnp.dot(o1, h2_ref[...],
                                 preferred_element_type=jnp.float32)
                         + hb2_ref[...], 0.0)
        o_ref[...] = jnp.dot(o2, h3_ref[...],
                             preferred_element_type=jnp.float32) + hb3_ref[...]

    return pl.pallas_call(
        body,
        out_shape=jax.ShapeDtypeStruct((B, 128), jnp.float32),
    )(sums, cnts, recipe, len_col, stats_p, baseline, emb_pad,
      Wi, Wh, bLr, S1p, sb1r, S2, sb2r,
      H1g, H1r, H1s, H1b, hb1r, H2, hb2r, H3p, hb3r)


def kernel(x, edge_index, edge_attr, batch, recipe, lengths, stats, baseline,
           W1, b1, W2, b2, We, be, V1, c1, V2, c2, emb, Wi, Wh, bL,
           S1, sb1, S2, sb2, H1, hb1, H2, hb2, H3, hb3):
    pad = EP - E
    src_p = jnp.concatenate([edge_index[0], jnp.zeros((pad,), jnp.int32)])
    dst_p = jnp.concatenate([edge_index[1],
                             jnp.full((pad,), N, jnp.int32)])
    attr_flat = jnp.concatenate([edge_attr.reshape(-1),
                                 jnp.zeros((pad * 5 + 16,), jnp.float32)])
    xpad = jnp.pad(x, ((0, 0), (0, 11)))
    zeros_acc = jnp.zeros((NACC, 16), jnp.float32)

    a0, a1 = _sc_conv1(src_p, dst_p, attr_flat, xpad, zeros_acc)

    W1p = jnp.pad(W1, ((0, 11), (0, 0)))
    h = _mlp1(xpad, a0, a1, W1p, b1.reshape(1, 128), W2, b2.reshape(1, 128))

    h_cm = h.reshape(N, 8, 16).transpose(1, 0, 2).reshape(8 * N, 16)
    We_cm = We.reshape(5, 8, 16).transpose(1, 0, 2)
    be_cm = be.reshape(8, 16)
    agg2_cm = _sc_conv2(src_p, dst_p, attr_flat, h_cm, We_cm, be_cm,
                        zeros_acc)
    agg2 = agg2_cm.transpose(1, 0, 2).reshape(N, 128)

    batch3 = batch.reshape(NGRID, 1, NBLK)
    sums, cnts = _mlp2pool(h, agg2, batch3, V1, c1.reshape(1, 128),
                           V2, c2.reshape(1, 128))

    emb_pad = jnp.pad(emb, ((0, VOCABP - (emb.shape[0])), (0, 0)))
    stats_p = jnp.pad(stats, ((0, 0), (0, 2)))
    S1p = jnp.pad(S1, ((0, 2), (0, 0)))
    H1g = H1[0:128]
    H1r = H1[128:192]
    H1s = H1[192:224]
    H1b = H1[224:225]
    H3p = jnp.pad(H3, ((0, 0), (0, 127)))
    hb3r = jnp.pad(hb3, (0, 127)).reshape(1, 128)

    out128 = _final(sums, cnts, recipe, lengths.reshape(B, 1),
                    stats_p, baseline, emb_pad,
                    Wi, Wh, bL.reshape(1, 256),
                    S1p, sb1.reshape(1, 32), S2, sb2.reshape(1, 32),
                    H1g, H1r, H1s, H1b, hb1.reshape(1, 128),
                    H2, hb2.reshape(1, 64), H3p, hb3r)
    return out128[:, 0:1]


# speedup vs baseline: 2.4161x; 1.0028x over previous
import functools

import jax
import jax.numpy as jnp
from jax import lax
from jax.experimental import pallas as pl
from jax.experimental.pallas import tpu as pltpu
from jax.experimental.pallas import tpu_sc as plsc

N = 100000
E = 3200000
EBLK = 128
EP = 782 * 32 * EBLK
NACC = 100008
B = 64
L = 50
VOCABP = 1024
NBLK = 1000
NGRID = N // NBLK

_SC_MESH = dict(core_axis_name="c", subcore_axis_name="s")


NB5 = EBLK * 5


def _sc_conv1(src_p, dst_p, attr_flat, xpad, zeros_acc):
    mesh = plsc.VectorSubcoreMesh(**_SC_MESH)
    per_tile = EP // 32
    nblk = per_tile // EBLK

    @functools.partial(
        pl.kernel,
        out_type=(jax.ShapeDtypeStruct((N, 16), jnp.float32),
                  jax.ShapeDtypeStruct((N, 16), jnp.float32)),
        mesh=mesh,
        scratch_types=[
            pltpu.VMEM((EBLK,), jnp.int32),
            pltpu.VMEM((EBLK,), jnp.int32),
            pltpu.VMEM((EBLK,), jnp.int32),
            pltpu.VMEM((EBLK,), jnp.int32),
            pltpu.VMEM((NB5 + 16,), jnp.float32),
            pltpu.VMEM((NB5 + 16,), jnp.float32),
            pltpu.VMEM((EBLK, 16), jnp.float32),
            pltpu.VMEM((EBLK, 16), jnp.float32),
            pltpu.VMEM((EBLK, 16), jnp.float32),
            pltpu.VMEM((EBLK, 16), jnp.float32),
            pltpu.VMEM_SHARED((NACC, 16), jnp.float32),
            pltpu.SemaphoreType.DMA,
            pltpu.SemaphoreType.DMA,
            pltpu.SemaphoreType.DMA,
            pltpu.SemaphoreType.DMA,
        ],
        compiler_params=pltpu.CompilerParams(use_tc_tiling_on_sc=False, needs_layout_passes=False),
    )
    def k(src_h, dst_h, attr_h, x_h, z_h, out0, out1,
          sb0, sb1, db0, db1, ab0, ab1, rb0, rb1, mb0, mb1, acc,
          sl0, sl1, sg0, sg1):
        c = lax.axis_index("c")
        s = lax.axis_index("s")
        S = [sb0, sb1]
        D = [db0, db1]
        A = [ab0, ab1]
        R = [rb0, rb1]
        M = [mb0, mb1]
        SL = [sl0, sl1]
        SG = [sg0, sg1]

        @pl.when(s == 0)
        def _():
            pltpu.sync_copy(z_h, acc)
        plsc.subcore_barrier()

        base = (c * 16 + s) * per_tile
        fmask = lax.broadcasted_iota(jnp.int32, (16,), 0) < 5

        def lin_start(i, b):
            e0 = base + jnp.minimum(i, nblk - 1) * EBLK
            pltpu.async_copy(src_h.at[pl.ds(e0, EBLK)], S[b], SL[b])
            pltpu.async_copy(dst_h.at[pl.ds(e0, EBLK)], D[b], SL[b])
            pltpu.async_copy(attr_h.at[pl.ds(e0 * 5, NB5)],
                             A[b].at[pl.ds(0, NB5)], SL[b])

        def lin_wait(b):
            pltpu.make_async_copy(src_h.at[pl.ds(0, EBLK)], S[b],
                                  SL[b]).wait()
            pltpu.make_async_copy(dst_h.at[pl.ds(0, EBLK)], D[b],
                                  SL[b]).wait()
            pltpu.make_async_copy(attr_h.at[pl.ds(0, NB5)],
                                  A[b].at[pl.ds(0, NB5)], SL[b]).wait()

        def g_start(b):
            pltpu.async_copy(x_h.at[S[b]], R[b], SG[b])

        def g_wait(b):
            pltpu.make_async_copy(x_h.at[S[b]], R[b], SG[b]).wait()

        def compute(b):
            rows = R[b]
            msg = M[b]
            attrb = A[b]

            def inner(e, carry2):
                a = attrb[pl.ds(e * 5, 16)]
                a = jnp.where(fmask, a, 0.0)
                msg[e] = jnp.maximum(rows[e] + a, 0.0)
                return carry2

            lax.fori_loop(0, EBLK, inner, 0, unroll=4)

        lin_start(0, 0)
        lin_wait(0)
        g_start(0)
        lin_start(1, 1)

        def pair(j, carry):
            for b in (0, 1):
                i = 2 * j + b
                lin_wait(1 - b)
                g_start(1 - b)
                g_wait(b)
                compute(b)
                pltpu.sync_copy(M[b], acc.at[D[b]], add=True)
                lin_start(i + 2, b)
            return carry

        lax.fori_loop(0, nblk // 2, pair, 0)
        g_wait(0)
        lin_wait(1)
        plsc.subcore_barrier()

        @pl.when(jnp.logical_and(s == 0, c == 0))
        def _():
            pltpu.sync_copy(acc.at[pl.ds(0, N)], out0)

        @pl.when(jnp.logical_and(s == 0, c == 1))
        def _():
            pltpu.sync_copy(acc.at[pl.ds(0, N)], out1)

    return k(src_p, dst_p, attr_flat, xpad, zeros_acc)


def _sc_conv2(src_p, dst_p, attr_flat, h_cm, We_cm, be_cm, zeros_acc):
    mesh = plsc.VectorSubcoreMesh(**_SC_MESH)
    per_tile = EP // 16
    nblk = per_tile // EBLK

    @functools.partial(
        pl.kernel,
        out_type=jax.ShapeDtypeStruct((8, N, 16), jnp.float32),
        mesh=mesh,
        scratch_types=[
            pltpu.VMEM((EBLK,), jnp.int32),
            pltpu.VMEM((EBLK,), jnp.int32),
            pltpu.VMEM((EBLK,), jnp.int32),
            pltpu.VMEM((EBLK,), jnp.int32),
            pltpu.VMEM((EBLK,), jnp.int32),
            pltpu.VMEM((EBLK,), jnp.int32),
            pltpu.VMEM((NB5 + 16,), jnp.float32),
            pltpu.VMEM((NB5 + 16,), jnp.float32),
            pltpu.VMEM((EBLK, 16), jnp.float32),
            pltpu.VMEM((EBLK, 16), jnp.float32),
            pltpu.VMEM((EBLK, 16), jnp.float32),
            pltpu.VMEM((EBLK, 16), jnp.float32),
            pltpu.VMEM((5, 16), jnp.float32),
            pltpu.VMEM((16,), jnp.float32),
            pltpu.VMEM_SHARED((NACC, 16), jnp.float32),
            pltpu.SemaphoreType.DMA,
            pltpu.SemaphoreType.DMA,
            pltpu.SemaphoreType.DMA,
            pltpu.SemaphoreType.DMA,
        ],
        compiler_params=pltpu.CompilerParams(use_tc_tiling_on_sc=False, needs_layout_passes=False),
    )
    def k(src_h, dst_h, attr_h, hcm_h, we_h, be_h, z_h, out,
          sb0, sb1, db0, db1, ib0, ib1, ab0, ab1, rb0, rb1, mb0, mb1,
          wev, bev, acc, sl0, sl1, sg0, sg1):
        c = lax.axis_index("c")
        s = lax.axis_index("s")
        base = s * per_tile
        S = [sb0, sb1]
        D = [db0, db1]
        I = [ib0, ib1]
        A = [ab0, ab1]
        R = [rb0, rb1]
        M = [mb0, mb1]
        SL = [sl0, sl1]
        SG = [sg0, sg1]

        def lin_start(i, b):
            e0 = base + jnp.minimum(i, nblk - 1) * EBLK
            pltpu.async_copy(src_h.at[pl.ds(e0, EBLK)], S[b], SL[b])
            pltpu.async_copy(dst_h.at[pl.ds(e0, EBLK)], D[b], SL[b])
            pltpu.async_copy(attr_h.at[pl.ds(e0 * 5, NB5)],
                             A[b].at[pl.ds(0, NB5)], SL[b])

        def lin_wait(b):
            pltpu.make_async_copy(src_h.at[pl.ds(0, EBLK)], S[b],
                                  SL[b]).wait()
            pltpu.make_async_copy(dst_h.at[pl.ds(0, EBLK)], D[b],
                                  SL[b]).wait()
            pltpu.make_async_copy(attr_h.at[pl.ds(0, NB5)],
                                  A[b].at[pl.ds(0, NB5)], SL[b]).wait()

        def g_start(b, off):
            pltpu.async_copy(hcm_h.at[pl.ds(off, N)].at[S[b]], R[b], SG[b])

        def g_wait(b):
            pltpu.make_async_copy(hcm_h.at[pl.ds(0, N)].at[S[b]], R[b],
                                  SG[b]).wait()

        for p in range(4):
            @pl.when(s == 0)
            def _():
                pltpu.sync_copy(z_h, acc)

            @pl.when(c == 0)
            def _():
                pltpu.sync_copy(we_h.at[2 * p], wev)
                pltpu.sync_copy(be_h.at[2 * p], bev)

            @pl.when(c == 1)
            def _():
                pltpu.sync_copy(we_h.at[2 * p + 1], wev)
                pltpu.sync_copy(be_h.at[2 * p + 1], bev)
            plsc.subcore_barrier()

            off = (2 * p + c) * N
            we0 = wev[0]
            we1 = wev[1]
            we2 = wev[2]
            we3 = wev[3]
            we4 = wev[4]
            bevv = bev[pl.ds(0, 16)]
            c5 = jnp.full((16,), 5, jnp.int32)
            iv_init = tuple(
                jnp.full((16,), kk, jnp.int32) for kk in range(5))

            def compute(b):
                rows = R[b]
                msg = M[b]
                attrb = A[b]

                def inner(e, carry2):
                    j0, j1, j2, j3, j4 = carry2
                    a0 = plsc.load_gather(attrb, [j0])
                    a1 = plsc.load_gather(attrb, [j1])
                    a2 = plsc.load_gather(attrb, [j2])
                    a3 = plsc.load_gather(attrb, [j3])
                    a4 = plsc.load_gather(attrb, [j4])
                    v = rows[e] + bevv + a0 * we0 + a1 * we1 \
                        + a2 * we2 + a3 * we3 + a4 * we4
                    msg[e] = jnp.maximum(v, 0.0)
                    return (j0 + c5, j1 + c5, j2 + c5, j3 + c5, j4 + c5)

                lax.fori_loop(0, EBLK, inner, iv_init, unroll=16)

            lin_start(0, 0)
            lin_wait(0)
            g_start(0, off)
            lin_start(1, 1)

            def pair(j, carry):
                for b in (0, 1):
                    i = 2 * j + b
                    lin_wait(1 - b)
                    g_start(1 - b, off)
                    g_wait(b)
                    compute(b)
                    pltpu.sync_copy(M[b], acc.at[D[b]], add=True)
                    lin_start(i + 2, b)
                return carry

            lax.fori_loop(0, nblk // 2, pair, 0)
            g_wait(0)
            lin_wait(1)
            plsc.subcore_barrier()

            @pl.when(jnp.logical_and(s == 0, c == 0))
            def _():
                pltpu.sync_copy(acc.at[pl.ds(0, N)], out.at[2 * p])

            @pl.when(jnp.logical_and(s == 0, c == 1))
            def _():
                pltpu.sync_copy(acc.at[pl.ds(0, N)], out.at[2 * p + 1])
            plsc.subcore_barrier()

    return k(src_p, dst_p, attr_flat, h_cm, We_cm, be_cm, zeros_acc)


def _mlp1(xpad, a0, a1, W1p, b1r, W2, b2r):
    def body(x_ref, a0_ref, a1_ref, w1_ref, b1_ref, w2_ref, b2_ref, o_ref):
        X = x_ref[...] + a0_ref[...] + a1_ref[...]
        t = jnp.maximum(jnp.dot(X, w1_ref[...],
                                preferred_element_type=jnp.float32)
                        + b1_ref[...], 0.0)
        o_ref[...] = jnp.maximum(jnp.dot(t, w2_ref[...],
                                         preferred_element_type=jnp.float32)
                                 + b2_ref[...], 0.0)

    return pl.pallas_call(
        body,
        grid=(NGRID,),
        in_specs=[
            pl.BlockSpec((NBLK, 16), lambda i: (i, 0)),
            pl.BlockSpec((NBLK, 16), lambda i: (i, 0)),
            pl.BlockSpec((NBLK, 16), lambda i: (i, 0)),
            pl.BlockSpec((16, 128), lambda i: (0, 0)),
            pl.BlockSpec((1, 128), lambda i: (0, 0)),
            pl.BlockSpec((128, 128), lambda i: (0, 0)),
            pl.BlockSpec((1, 128), lambda i: (0, 0)),
        ],
        out_specs=pl.BlockSpec((NBLK, 128), lambda i: (i, 0)),
        out_shape=jax.ShapeDtypeStruct((N, 128), jnp.float32),
    )(xpad, a0, a1, W1p, b1r, W2, b2r)


def _mlp2pool(h, agg2, batch3, V1, c1r, V2, c2r):
    def body(h_ref, a_ref, b_ref, v1_ref, c1_ref, v2_ref, c2_ref,
             sums_ref, cnts_ref):
        @pl.when(pl.program_id(0) == 0)
        def _():
            sums_ref[...] = jnp.zeros_like(sums_ref)
            cnts_ref[...] = jnp.zeros_like(cnts_ref)

        X = h_ref[...] + a_ref[...]
        t = jnp.maximum(jnp.dot(X, v1_ref[...],
                                preferred_element_type=jnp.float32)
                        + c1_ref[...], 0.0)
        h2 = jnp.maximum(jnp.dot(t, v2_ref[...],
                                 preferred_element_type=jnp.float32)
                         + c2_ref[...], 0.0)
        bb = b_ref[...].reshape(1, NBLK)
        ohT = (jnp.broadcast_to(bb, (B, NBLK))
               == lax.broadcasted_iota(jnp.int32, (B, NBLK), 0)
               ).astype(jnp.float32)
        sums_ref[...] += jnp.dot(ohT, h2, preferred_element_type=jnp.float32)
        cnt = jnp.sum(ohT, axis=1, keepdims=True)
        cnts_ref[...] += jnp.broadcast_to(cnt, (B, 128))

    return pl.pallas_call(
        body,
        grid=(NGRID,),
        in_specs=[
            pl.BlockSpec((NBLK, 128), lambda i: (i, 0)),
            pl.BlockSpec((NBLK, 128), lambda i: (i, 0)),
            pl.BlockSpec((1, 1, NBLK), lambda i: (i, 0, 0)),
            pl.BlockSpec((128, 128), lambda i: (0, 0)),
            pl.BlockSpec((1, 128), lambda i: (0, 0)),
            pl.BlockSpec((128, 128), lambda i: (0, 0)),
            pl.BlockSpec((1, 128), lambda i: (0, 0)),
        ],
        out_specs=[
            pl.BlockSpec((B, 128), lambda i: (0, 0)),
            pl.BlockSpec((B, 128), lambda i: (0, 0)),
        ],
        out_shape=[
            jax.ShapeDtypeStruct((B, 128), jnp.float32),
            jax.ShapeDtypeStruct((B, 128), jnp.float32),
        ],
        compiler_params=pltpu.CompilerParams(
            dimension_semantics=("arbitrary",)),
    )(h, agg2, batch3, V1, c1r, V2, c2r)


def _final(sums, cnts, recipe, len_col, stats_p, baseline, emb_pad,
           Wi, Wh, bLr, S1p, sb1r, S2, sb2r,
           H1g, H1r, H1s, H1b, hb1r, H2, hb2r, H3p, hb3r):
    def body(sums_ref, cnts_ref, rec_ref, len_ref, st_ref, base_ref, emb_ref,
             wi_ref, wh_ref, bl_ref, s1_ref, sb1_ref, s2_ref, sb2_ref,
             h1g_ref, h1r_ref, h1s_ref, h1b_ref, hb1_ref, h2_ref, hb2_ref,
             h3_ref, hb3_ref, o_ref):
        g = sums_ref[...] / jnp.maximum(cnts_ref[...], 1.0)

        idxc = jnp.clip(len_ref[...] - 1, 0, L - 1)
        wi = wi_ref[...]
        wh = wh_ref[...]
        bl = bl_ref[...]
        emb = emb_ref[...]

        def sigmoid(v):
            return 1.0 / (1.0 + jnp.exp(-v))

        rec = rec_ref[...]
        zero = jnp.zeros((B, 64), jnp.float32)
        hh, cc, sel = zero, zero, zero
        for t in range(L):
            rt = rec[:, t:t + 1]
            oh = (jnp.broadcast_to(rt, (B, VOCABP))
                  == lax.broadcasted_iota(jnp.int32, (B, VOCABP), 1)
                  ).astype(jnp.float32)
            xt = jnp.dot(oh, emb, preferred_element_type=jnp.float32)
            z = (jnp.dot(xt, wi, preferred_element_type=jnp.float32)
                 + jnp.dot(hh, wh, preferred_element_type=jnp.float32) + bl)
            i_ = sigmoid(z[:, 0:64])
            f_ = sigmoid(z[:, 64:128])
            g_ = jnp.tanh(z[:, 128:192])
            o_ = sigmoid(z[:, 192:256])
            cc = f_ * cc + i_ * g_
            hh = o_ * jnp.tanh(cc)
            sel = jnp.where(idxc == t, hh, sel)
        r = sel

        st = jnp.maximum(jnp.dot(st_ref[...], s1_ref[...],
                                 preferred_element_type=jnp.float32)
                         + sb1_ref[...], 0.0)
        s = jnp.dot(st, s2_ref[...], preferred_element_type=jnp.float32) \
            + sb2_ref[...]

        z1 = (jnp.dot(g, h1g_ref[...], preferred_element_type=jnp.float32)
              + jnp.dot(r, h1r_ref[...], preferred_element_type=jnp.float32)
              + jnp.dot(s, h1s_ref[...], preferred_element_type=jnp.float32)
              + base_ref[...] * h1b_ref[...]
              + hb1_ref[...])
        o1 = jnp.maximum(z1, 0.0)
        o2 = jnp.maximum(jnp.dot(o1, h2_ref[...],
                                 preferred_element_type=jnp.float32)
                         + hb2_ref[...], 0.0)
        o_ref[...] = jnp.dot(o2, h3_ref[...],
                             preferred_element_type=jnp.float32) + hb3_ref[...]

    return pl.pallas_call(
        body,
        out_shape=jax.ShapeDtypeStruct((B, 128), jnp.float32),
    )(sums, cnts, recipe, len_col, stats_p, baseline, emb_pad,
      Wi, Wh, bLr, S1p, sb1r, S2, sb2r,
      H1g, H1r, H1s, H1b, hb1r, H2, hb2r, H3p, hb3r)


def kernel(x, edge_index, edge_attr, batch, recipe, lengths, stats, baseline,
           W1, b1, W2, b2, We, be, V1, c1, V2, c2, emb, Wi, Wh, bL,
           S1, sb1, S2, sb2, H1, hb1, H2, hb2, H3, hb3):
    pad = EP - E
    src_p = jnp.concatenate([edge_index[0], jnp.zeros((pad,), jnp.int32)])
    dst_p = jnp.concatenate([edge_index[1],
                             jnp.full((pad,), N, jnp.int32)])
    attr_flat = jnp.concatenate([edge_attr.reshape(-1),
                                 jnp.zeros((pad * 5 + 16,), jnp.float32)])
    xpad = jnp.pad(x, ((0, 0), (0, 11)))
    zeros_acc = jnp.zeros((NACC, 16), jnp.float32)

    a0, a1 = _sc_conv1(src_p, dst_p, attr_flat, xpad, zeros_acc)

    W1p = jnp.pad(W1, ((0, 11), (0, 0)))
    h = _mlp1(xpad, a0, a1, W1p, b1.reshape(1, 128), W2, b2.reshape(1, 128))

    h_cm = h.reshape(N, 8, 16).transpose(1, 0, 2).reshape(8 * N, 16)
    We_cm = We.reshape(5, 8, 16).transpose(1, 0, 2)
    be_cm = be.reshape(8, 16)
    agg2_cm = _sc_conv2(src_p, dst_p, attr_flat, h_cm, We_cm, be_cm,
                        zeros_acc)
    agg2 = agg2_cm.transpose(1, 0, 2).reshape(N, 128)

    batch3 = batch.reshape(NGRID, 1, NBLK)
    sums, cnts = _mlp2pool(h, agg2, batch3, V1, c1.reshape(1, 128),
                           V2, c2.reshape(1, 128))

    emb_pad = jnp.pad(emb, ((0, VOCABP - (emb.shape[0])), (0, 0)))
    stats_p = jnp.pad(stats, ((0, 0), (0, 2)))
    S1p = jnp.pad(S1, ((0, 2), (0, 0)))
    H1g = H1[0:128]
    H1r = H1[128:192]
    H1s = H1[192:224]
    H1b = H1[224:225]
    H3p = jnp.pad(H3, ((0, 0), (0, 127)))
    hb3r = jnp.pad(hb3, (0, 127)).reshape(1, 128)

    out128 = _final(sums, cnts, recipe, lengths.reshape(B, 1),
                    stats_p, baseline, emb_pad,
                    Wi, Wh, bL.reshape(1, 256),
                    S1p, sb1.reshape(1, 32), S2, sb2.reshape(1, 32),
                    H1g, H1r, H1s, H1b, hb1.reshape(1, 128),
                    H2, hb2.reshape(1, 64), H3p, hb3r)
    return out128[:, 0:1]


# speedup vs baseline: 2.5490x; 1.0550x over previous
import functools

import jax
import jax.numpy as jnp
from jax import lax
from jax.experimental import pallas as pl
from jax.experimental.pallas import tpu as pltpu
from jax.experimental.pallas import tpu_sc as plsc

N = 100000
E = 3200000
EBLK = 128
EP = 784 * 32 * EBLK
NACC = 100008
B = 64
L = 50
VOCABP = 1024
NBLK = 1000
NGRID = N // NBLK

_SC_MESH = dict(core_axis_name="c", subcore_axis_name="s")


NB5 = EBLK * 5


def _sc_conv1(src_p, dst_p, attr_flat, xpad, zeros_acc):
    mesh = plsc.VectorSubcoreMesh(**_SC_MESH)
    per_tile = EP // 32
    nblk = per_tile // EBLK

    @functools.partial(
        pl.kernel,
        out_type=(jax.ShapeDtypeStruct((N, 16), jnp.float32),
                  jax.ShapeDtypeStruct((N, 16), jnp.float32)),
        mesh=mesh,
        scratch_types=[
            pltpu.VMEM((EBLK,), jnp.int32),
            pltpu.VMEM((EBLK,), jnp.int32),
            pltpu.VMEM((EBLK,), jnp.int32),
            pltpu.VMEM((EBLK,), jnp.int32),
            pltpu.VMEM((EBLK,), jnp.int32),
            pltpu.VMEM((EBLK,), jnp.int32),
            pltpu.VMEM((NB5 + 16,), jnp.float32),
            pltpu.VMEM((NB5 + 16,), jnp.float32),
            pltpu.VMEM((EBLK, 16), jnp.float32),
            pltpu.VMEM((EBLK, 16), jnp.float32),
            pltpu.VMEM((EBLK, 16), jnp.float32),
            pltpu.VMEM((EBLK, 16), jnp.float32),
            pltpu.VMEM_SHARED((NACC, 16), jnp.float32),
            pltpu.SemaphoreType.DMA,
            pltpu.SemaphoreType.DMA,
            pltpu.SemaphoreType.DMA,
            pltpu.SemaphoreType.DMA,
            pltpu.SemaphoreType.DMA,
            pltpu.SemaphoreType.DMA,
        ],
        compiler_params=pltpu.CompilerParams(use_tc_tiling_on_sc=False, needs_layout_passes=False),
    )
    def k(src_h, dst_h, attr_h, x_h, z_h, out0, out1,
          sb0, sb1, db0, db1, db2, db3, ab0, ab1, rb0, rb1, mb0, mb1, acc,
          sl0, sl1, sg0, sg1, ss0, ss1):
        c = lax.axis_index("c")
        s = lax.axis_index("s")
        S = [sb0, sb1]
        D = [db0, db1, db2, db3]
        A = [ab0, ab1]
        R = [rb0, rb1]
        M = [mb0, mb1]
        SL = [sl0, sl1]
        SG = [sg0, sg1]
        SS = [ss0, ss1]

        @pl.when(s == 0)
        def _():
            pltpu.sync_copy(z_h, acc)
        plsc.subcore_barrier()

        base = (c * 16 + s) * per_tile
        fmask = lax.broadcasted_iota(jnp.int32, (16,), 0) < 5

        def lin_start(i, lb, db):
            e0 = base + jnp.minimum(i, nblk - 1) * EBLK
            pltpu.async_copy(src_h.at[pl.ds(e0, EBLK)], S[lb], SL[lb])
            pltpu.async_copy(dst_h.at[pl.ds(e0, EBLK)], D[db], SL[lb])
            pltpu.async_copy(attr_h.at[pl.ds(e0 * 5, NB5)],
                             A[lb].at[pl.ds(0, NB5)], SL[lb])

        def lin_wait(lb, db):
            pltpu.make_async_copy(src_h.at[pl.ds(0, EBLK)], S[lb],
                                  SL[lb]).wait()
            pltpu.make_async_copy(dst_h.at[pl.ds(0, EBLK)], D[db],
                                  SL[lb]).wait()
            pltpu.make_async_copy(attr_h.at[pl.ds(0, NB5)],
                                  A[lb].at[pl.ds(0, NB5)], SL[lb]).wait()

        def g_start(lb):
            pltpu.async_copy(x_h.at[S[lb]], R[lb], SG[lb])

        def g_wait(lb):
            pltpu.make_async_copy(x_h.at[S[lb]], R[lb], SG[lb]).wait()

        def scat_start(lb, db):
            pltpu.async_copy(M[lb], acc.at[D[db]], SS[lb], add=True)

        def scat_wait(lb, db):
            pltpu.make_async_copy(M[lb], acc.at[D[db]], SS[lb]).wait()

        def compute(b):
            rows = R[b]
            msg = M[b]
            attrb = A[b]

            def inner(e, carry2):
                a = attrb[pl.ds(e * 5, 16)]
                a = jnp.where(fmask, a, 0.0)
                msg[e] = jnp.maximum(rows[e] + a, 0.0)
                return carry2

            lax.fori_loop(0, EBLK, inner, 0, unroll=4)

        lin_start(0, 0, 0)
        lin_wait(0, 0)
        g_start(0)
        lin_start(1, 1, 1)

        def quad(j, carry):
            for b in range(4):
                i = 4 * j + b
                lb = b % 2
                lin_wait(1 - lb, (b + 1) % 4)
                g_start(1 - lb)
                g_wait(lb)

                @pl.when(i >= 2)
                def _():
                    scat_wait(lb, (b + 2) % 4)
                compute(lb)
                scat_start(lb, b)
                lin_start(i + 2, lb, (b + 2) % 4)
            return carry

        lax.fori_loop(0, nblk // 4, quad, 0)
        g_wait(0)
        lin_wait(1, 1)
        scat_wait(0, 2)
        scat_wait(1, 3)
        plsc.subcore_barrier()

        @pl.when(jnp.logical_and(s == 0, c == 0))
        def _():
            pltpu.sync_copy(acc.at[pl.ds(0, N)], out0)

        @pl.when(jnp.logical_and(s == 0, c == 1))
        def _():
            pltpu.sync_copy(acc.at[pl.ds(0, N)], out1)

    return k(src_p, dst_p, attr_flat, xpad, zeros_acc)


def _sc_conv2(src_p, dst_p, attr_flat, h_cm, We_cm, be_cm, zeros_acc):
    mesh = plsc.VectorSubcoreMesh(**_SC_MESH)
    per_tile = EP // 16
    nblk = per_tile // EBLK

    @functools.partial(
        pl.kernel,
        out_type=jax.ShapeDtypeStruct((8, N, 16), jnp.float32),
        mesh=mesh,
        scratch_types=[
            pltpu.VMEM((EBLK,), jnp.int32),
            pltpu.VMEM((EBLK,), jnp.int32),
            pltpu.VMEM((EBLK,), jnp.int32),
            pltpu.VMEM((EBLK,), jnp.int32),
            pltpu.VMEM((EBLK,), jnp.int32),
            pltpu.VMEM((EBLK,), jnp.int32),
            pltpu.VMEM((NB5 + 16,), jnp.float32),
            pltpu.VMEM((NB5 + 16,), jnp.float32),
            pltpu.VMEM((EBLK, 16), jnp.float32),
            pltpu.VMEM((EBLK, 16), jnp.float32),
            pltpu.VMEM((EBLK, 16), jnp.float32),
            pltpu.VMEM((EBLK, 16), jnp.float32),
            pltpu.VMEM((5, 16), jnp.float32),
            pltpu.VMEM((16,), jnp.float32),
            pltpu.VMEM_SHARED((NACC, 16), jnp.float32),
            pltpu.SemaphoreType.DMA,
            pltpu.SemaphoreType.DMA,
            pltpu.SemaphoreType.DMA,
            pltpu.SemaphoreType.DMA,
            pltpu.SemaphoreType.DMA,
            pltpu.SemaphoreType.DMA,
        ],
        compiler_params=pltpu.CompilerParams(use_tc_tiling_on_sc=False, needs_layout_passes=False),
    )
    def k(src_h, dst_h, attr_h, hcm_h, we_h, be_h, z_h, out,
          sb0, sb1, db0, db1, db2, db3, ab0, ab1, rb0, rb1, mb0, mb1,
          wev, bev, acc, sl0, sl1, sg0, sg1, ss0, ss1):
        c = lax.axis_index("c")
        s = lax.axis_index("s")
        base = s * per_tile
        S = [sb0, sb1]
        D = [db0, db1, db2, db3]
        A = [ab0, ab1]
        R = [rb0, rb1]
        M = [mb0, mb1]
        SL = [sl0, sl1]
        SG = [sg0, sg1]
        SS = [ss0, ss1]

        def lin_start(i, lb, db):
            e0 = base + jnp.minimum(i, nblk - 1) * EBLK
            pltpu.async_copy(src_h.at[pl.ds(e0, EBLK)], S[lb], SL[lb])
            pltpu.async_copy(dst_h.at[pl.ds(e0, EBLK)], D[db], SL[lb])
            pltpu.async_copy(attr_h.at[pl.ds(e0 * 5, NB5)],
                             A[lb].at[pl.ds(0, NB5)], SL[lb])

        def lin_wait(lb, db):
            pltpu.make_async_copy(src_h.at[pl.ds(0, EBLK)], S[lb],
                                  SL[lb]).wait()
            pltpu.make_async_copy(dst_h.at[pl.ds(0, EBLK)], D[db],
                                  SL[lb]).wait()
            pltpu.make_async_copy(attr_h.at[pl.ds(0, NB5)],
                                  A[lb].at[pl.ds(0, NB5)], SL[lb]).wait()

        def g_start(lb, off):
            pltpu.async_copy(hcm_h.at[pl.ds(off, N)].at[S[lb]], R[lb],
                             SG[lb])

        def g_wait(lb):
            pltpu.make_async_copy(hcm_h.at[pl.ds(0, N)].at[S[lb]], R[lb],
                                  SG[lb]).wait()

        def scat_start(lb, db):
            pltpu.async_copy(M[lb], acc.at[D[db]], SS[lb], add=True)

        def scat_wait(lb, db):
            pltpu.make_async_copy(M[lb], acc.at[D[db]], SS[lb]).wait()

        for p in range(4):
            @pl.when(s == 0)
            def _():
                pltpu.sync_copy(z_h, acc)

            @pl.when(c == 0)
            def _():
                pltpu.sync_copy(we_h.at[2 * p], wev)
                pltpu.sync_copy(be_h.at[2 * p], bev)

            @pl.when(c == 1)
            def _():
                pltpu.sync_copy(we_h.at[2 * p + 1], wev)
                pltpu.sync_copy(be_h.at[2 * p + 1], bev)
            plsc.subcore_barrier()

            off = (2 * p + c) * N
            we0 = wev[0]
            we1 = wev[1]
            we2 = wev[2]
            we3 = wev[3]
            we4 = wev[4]
            bevv = bev[pl.ds(0, 16)]
            c5 = jnp.full((16,), 5, jnp.int32)
            iv_init = tuple(
                jnp.full((16,), kk, jnp.int32) for kk in range(5))

            def compute(b):
                rows = R[b]
                msg = M[b]
                attrb = A[b]

                def inner(e, carry2):
                    j0, j1, j2, j3, j4 = carry2
                    a0 = plsc.load_gather(attrb, [j0])
                    a1 = plsc.load_gather(attrb, [j1])
                    a2 = plsc.load_gather(attrb, [j2])
                    a3 = plsc.load_gather(attrb, [j3])
                    a4 = plsc.load_gather(attrb, [j4])
                    v = rows[e] + bevv + a0 * we0 + a1 * we1 \
                        + a2 * we2 + a3 * we3 + a4 * we4
                    msg[e] = jnp.maximum(v, 0.0)
                    return (j0 + c5, j1 + c5, j2 + c5, j3 + c5, j4 + c5)

                lax.fori_loop(0, EBLK, inner, iv_init, unroll=4)

            lin_start(0, 0, 0)
            lin_wait(0, 0)
            g_start(0, off)
            lin_start(1, 1, 1)

            def quad(j, carry):
                for b in range(4):
                    i = 4 * j + b
                    lb = b % 2
                    lin_wait(1 - lb, (b + 1) % 4)
                    g_start(1 - lb, off)
                    g_wait(lb)

                    @pl.when(i >= 2)
                    def _():
                        scat_wait(lb, (b + 2) % 4)
                    compute(lb)
                    scat_start(lb, b)
                    lin_start(i + 2, lb, (b + 2) % 4)
                return carry

            lax.fori_loop(0, nblk // 4, quad, 0)
            g_wait(0)
            lin_wait(1, 1)
            scat_wait(0, 2)
            scat_wait(1, 3)
            plsc.subcore_barrier()

            @pl.when(jnp.logical_and(s == 0, c == 0))
            def _():
                pltpu.sync_copy(acc.at[pl.ds(0, N)], out.at[2 * p])

            @pl.when(jnp.logical_and(s == 0, c == 1))
            def _():
                pltpu.sync_copy(acc.at[pl.ds(0, N)], out.at[2 * p + 1])
            plsc.subcore_barrier()

    return k(src_p, dst_p, attr_flat, h_cm, We_cm, be_cm, zeros_acc)


def _mlp1(xpad, a0, a1, W1p, b1r, W2, b2r):
    def body(x_ref, a0_ref, a1_ref, w1_ref, b1_ref, w2_ref, b2_ref, o_ref):
        X = x_ref[...] + a0_ref[...] + a1_ref[...]
        t = jnp.maximum(jnp.dot(X, w1_ref[...],
                                preferred_element_type=jnp.float32)
                        + b1_ref[...], 0.0)
        o_ref[...] = jnp.maximum(jnp.dot(t, w2_ref[...],
                                         preferred_element_type=jnp.float32)
                                 + b2_ref[...], 0.0)

    return pl.pallas_call(
        body,
        grid=(NGRID,),
        in_specs=[
            pl.BlockSpec((NBLK, 16), lambda i: (i, 0)),
            pl.BlockSpec((NBLK, 16), lambda i: (i, 0)),
            pl.BlockSpec((NBLK, 16), lambda i: (i, 0)),
            pl.BlockSpec((16, 128), lambda i: (0, 0)),
            pl.BlockSpec((1, 128), lambda i: (0, 0)),
            pl.BlockSpec((128, 128), lambda i: (0, 0)),
            pl.BlockSpec((1, 128), lambda i: (0, 0)),
        ],
        out_specs=pl.BlockSpec((NBLK, 128), lambda i: (i, 0)),
        out_shape=jax.ShapeDtypeStruct((N, 128), jnp.float32),
    )(xpad, a0, a1, W1p, b1r, W2, b2r)


def _mlp2pool(h, agg2, batch3, V1, c1r, V2, c2r):
    def body(h_ref, a_ref, b_ref, v1_ref, c1_ref, v2_ref, c2_ref,
             sums_ref, cnts_ref):
        @pl.when(pl.program_id(0) == 0)
        def _():
            sums_ref[...] = jnp.zeros_like(sums_ref)
            cnts_ref[...] = jnp.zeros_like(cnts_ref)

        X = h_ref[...] + a_ref[...]
        t = jnp.maximum(jnp.dot(X, v1_ref[...],
                                preferred_element_type=jnp.float32)
                        + c1_ref[...], 0.0)
        h2 = jnp.maximum(jnp.dot(t, v2_ref[...],
                                 preferred_element_type=jnp.float32)
                         + c2_ref[...], 0.0)
        bb = b_ref[...].reshape(1, NBLK)
        ohT = (jnp.broadcast_to(bb, (B, NBLK))
               == lax.broadcasted_iota(jnp.int32, (B, NBLK), 0)
               ).astype(jnp.float32)
        sums_ref[...] += jnp.dot(ohT, h2, preferred_element_type=jnp.float32)
        cnt = jnp.sum(ohT, axis=1, keepdims=True)
        cnts_ref[...] += jnp.broadcast_to(cnt, (B, 128))

    return pl.pallas_call(
        body,
        grid=(NGRID,),
        in_specs=[
            pl.BlockSpec((NBLK, 128), lambda i: (i, 0)),
            pl.BlockSpec((NBLK, 128), lambda i: (i, 0)),
            pl.BlockSpec((1, 1, NBLK), lambda i: (i, 0, 0)),
            pl.BlockSpec((128, 128), lambda i: (0, 0)),
            pl.BlockSpec((1, 128), lambda i: (0, 0)),
            pl.BlockSpec((128, 128), lambda i: (0, 0)),
            pl.BlockSpec((1, 128), lambda i: (0, 0)),
        ],
        out_specs=[
            pl.BlockSpec((B, 128), lambda i: (0, 0)),
            pl.BlockSpec((B, 128), lambda i: (0, 0)),
        ],
        out_shape=[
            jax.ShapeDtypeStruct((B, 128), jnp.float32),
            jax.ShapeDtypeStruct((B, 128), jnp.float32),
        ],
        compiler_params=pltpu.CompilerParams(
            dimension_semantics=("arbitrary",)),
    )(h, agg2, batch3, V1, c1r, V2, c2r)


def _final(sums, cnts, recipe, len_col, stats_p, baseline, emb_pad,
           Wi, Wh, bLr, S1p, sb1r, S2, sb2r,
           H1g, H1r, H1s, H1b, hb1r, H2, hb2r, H3p, hb3r):
    def body(sums_ref, cnts_ref, rec_ref, len_ref, st_ref, base_ref, emb_ref,
             wi_ref, wh_ref, bl_ref, s1_ref, sb1_ref, s2_ref, sb2_ref,
             h1g_ref, h1r_ref, h1s_ref, h1b_ref, hb1_ref, h2_ref, hb2_ref,
             h3_ref, hb3_ref, o_ref):
        g = sums_ref[...] / jnp.maximum(cnts_ref[...], 1.0)

        idxc = jnp.clip(len_ref[...] - 1, 0, L - 1)
        wi = wi_ref[...]
        wh = wh_ref[...]
        bl = bl_ref[...]
        emb = emb_ref[...]

        def sigmoid(v):
            return 1.0 / (1.0 + jnp.exp(-v))

        rec = rec_ref[...]
        zero = jnp.zeros((B, 64), jnp.float32)
        hh, cc, sel = zero, zero, zero
        for t in range(L):
            rt = rec[:, t:t + 1]
            oh = (jnp.broadcast_to(rt, (B, VOCABP))
                  == lax.broadcasted_iota(jnp.int32, (B, VOCABP), 1)
                  ).astype(jnp.float32)
            xt = jnp.dot(oh, emb, preferred_element_type=jnp.float32)
            z = (jnp.dot(xt, wi, preferred_element_type=jnp.float32)
                 + jnp.dot(hh, wh, preferred_element_type=jnp.float32) + bl)
            i_ = sigmoid(z[:, 0:64])
            f_ = sigmoid(z[:, 64:128])
            g_ = jnp.tanh(z[:, 128:192])
            o_ = sigmoid(z[:, 192:256])
            cc = f_ * cc + i_ * g_
            hh = o_ * jnp.tanh(cc)
            sel = jnp.where(idxc == t, hh, sel)
        r = sel

        st = jnp.maximum(jnp.dot(st_ref[...], s1_ref[...],
                                 preferred_element_type=jnp.float32)
                         + sb1_ref[...], 0.0)
        s = jnp.dot(st, s2_ref[...], preferred_element_type=jnp.float32) \
            + sb2_ref[...]

        z1 = (jnp.dot(g, h1g_ref[...], preferred_element_type=jnp.float32)
              + jnp.dot(r, h1r_ref[...], preferred_element_type=jnp.float32)
              + jnp.dot(s, h1s_ref[...], preferred_element_type=jnp.float32)
              + base_ref[...] * h1b_ref[...]
              + hb1_ref[...])
        o1 = jnp.maximum(z1, 0.0)
        o2 = jnp.maximum(jnp.dot(o1, h2_ref[...],
                                 preferred_element_type=jnp.float32)
                         + hb2_ref[...], 0.0)
        o_ref[...] = jnp.dot(o2, h3_ref[...],
                             preferred_element_type=jnp.float32) + hb3_ref[...]

    return pl.pallas_call(
        body,
        out_shape=jax.ShapeDtypeStruct((B, 128), jnp.float32),
    )(sums, cnts, recipe, len_col, stats_p, baseline, emb_pad,
      Wi, Wh, bLr, S1p, sb1r, S2, sb2r,
      H1g, H1r, H1s, H1b, hb1r, H2, hb2r, H3p, hb3r)


def kernel(x, edge_index, edge_attr, batch, recipe, lengths, stats, baseline,
           W1, b1, W2, b2, We, be, V1, c1, V2, c2, emb, Wi, Wh, bL,
           S1, sb1, S2, sb2, H1, hb1, H2, hb2, H3, hb3):
    pad = EP - E
    src_p = jnp.concatenate([edge_index[0], jnp.zeros((pad,), jnp.int32)])
    dst_p = jnp.concatenate([edge_index[1],
                             jnp.full((pad,), N, jnp.int32)])
    attr_flat = jnp.concatenate([edge_attr.reshape(-1),
                                 jnp.zeros((pad * 5 + 16,), jnp.float32)])
    xpad = jnp.pad(x, ((0, 0), (0, 11)))
    zeros_acc = jnp.zeros((NACC, 16), jnp.float32)

    a0, a1 = _sc_conv1(src_p, dst_p, attr_flat, xpad, zeros_acc)

    W1p = jnp.pad(W1, ((0, 11), (0, 0)))
    h = _mlp1(xpad, a0, a1, W1p, b1.reshape(1, 128), W2, b2.reshape(1, 128))

    h_cm = h.reshape(N, 8, 16).transpose(1, 0, 2).reshape(8 * N, 16)
    We_cm = We.reshape(5, 8, 16).transpose(1, 0, 2)
    be_cm = be.reshape(8, 16)
    agg2_cm = _sc_conv2(src_p, dst_p, attr_flat, h_cm, We_cm, be_cm,
                        zeros_acc)
    agg2 = agg2_cm.transpose(1, 0, 2).reshape(N, 128)

    batch3 = batch.reshape(NGRID, 1, NBLK)
    sums, cnts = _mlp2pool(h, agg2, batch3, V1, c1.reshape(1, 128),
                           V2, c2.reshape(1, 128))

    emb_pad = jnp.pad(emb, ((0, VOCABP - (emb.shape[0])), (0, 0)))
    stats_p = jnp.pad(stats, ((0, 0), (0, 2)))
    S1p = jnp.pad(S1, ((0, 2), (0, 0)))
    H1g = H1[0:128]
    H1r = H1[128:192]
    H1s = H1[192:224]
    H1b = H1[224:225]
    H3p = jnp.pad(H3, ((0, 0), (0, 127)))
    hb3r = jnp.pad(hb3, (0, 127)).reshape(1, 128)

    out128 = _final(sums, cnts, recipe, lengths.reshape(B, 1),
                    stats_p, baseline, emb_pad,
                    Wi, Wh, bL.reshape(1, 256),
                    S1p, sb1.reshape(1, 32), S2, sb2.reshape(1, 32),
                    H1g, H1r, H1s, H1b, hb1.reshape(1, 128),
                    H2, hb2.reshape(1, 64), H3p, hb3r)
    return out128[:, 0:1]


# speedup vs baseline: 4.1349x; 1.6221x over previous
import functools

import jax
import jax.numpy as jnp
from jax import lax
from jax.experimental import pallas as pl
from jax.experimental.pallas import tpu as pltpu
from jax.experimental.pallas import tpu_sc as plsc

N = 100000
E = 3200000
EBLK = 128
EP = 784 * 32 * EBLK
NACC = 100008
B = 64
L = 50
VOCABP = 1024
NBLK = 1000
NGRID = N // NBLK

_SC_MESH = dict(core_axis_name="c", subcore_axis_name="s")


NB5 = EBLK * 5


def _sc_conv1(src_p, dst_p, attr_flat, xpad, zeros_acc):
    mesh = plsc.VectorSubcoreMesh(**_SC_MESH)
    per_tile = EP // 32
    nblk = per_tile // EBLK

    @functools.partial(
        pl.kernel,
        out_type=(jax.ShapeDtypeStruct((N, 16), jnp.float32),
                  jax.ShapeDtypeStruct((N, 16), jnp.float32)),
        mesh=mesh,
        scratch_types=[
            pltpu.VMEM((EBLK,), jnp.int32),
            pltpu.VMEM((EBLK,), jnp.int32),
            pltpu.VMEM((EBLK,), jnp.int32),
            pltpu.VMEM((EBLK,), jnp.int32),
            pltpu.VMEM((EBLK,), jnp.int32),
            pltpu.VMEM((EBLK,), jnp.int32),
            pltpu.VMEM((NB5 + 16,), jnp.float32),
            pltpu.VMEM((NB5 + 16,), jnp.float32),
            pltpu.VMEM((EBLK, 16), jnp.float32),
            pltpu.VMEM((EBLK, 16), jnp.float32),
            pltpu.VMEM((EBLK, 16), jnp.float32),
            pltpu.VMEM((EBLK, 16), jnp.float32),
            pltpu.VMEM_SHARED((NACC, 16), jnp.float32),
            pltpu.SemaphoreType.DMA,
            pltpu.SemaphoreType.DMA,
            pltpu.SemaphoreType.DMA,
            pltpu.SemaphoreType.DMA,
            pltpu.SemaphoreType.DMA,
            pltpu.SemaphoreType.DMA,
        ],
        compiler_params=pltpu.CompilerParams(use_tc_tiling_on_sc=False, needs_layout_passes=False),
    )
    def k(src_h, dst_h, attr_h, x_h, z_h, out0, out1,
          sb0, sb1, db0, db1, db2, db3, ab0, ab1, rb0, rb1, mb0, mb1, acc,
          sl0, sl1, sg0, sg1, ss0, ss1):
        c = lax.axis_index("c")
        s = lax.axis_index("s")
        S = [sb0, sb1]
        D = [db0, db1, db2, db3]
        A = [ab0, ab1]
        R = [rb0, rb1]
        M = [mb0, mb1]
        SL = [sl0, sl1]
        SG = [sg0, sg1]
        SS = [ss0, ss1]

        @pl.when(s == 0)
        def _():
            pltpu.sync_copy(z_h, acc)
        plsc.subcore_barrier()

        base = (c * 16 + s) * per_tile
        fmask = lax.broadcasted_iota(jnp.int32, (16,), 0) < 5

        def lin_start(i, lb, db):
            e0 = base + jnp.minimum(i, nblk - 1) * EBLK
            pltpu.async_copy(src_h.at[pl.ds(e0, EBLK)], S[lb], SL[lb])
            pltpu.async_copy(dst_h.at[pl.ds(e0, EBLK)], D[db], SL[lb])
            pltpu.async_copy(attr_h.at[pl.ds(e0 * 5, NB5)],
                             A[lb].at[pl.ds(0, NB5)], SL[lb])

        def lin_wait(lb, db):
            pltpu.make_async_copy(src_h.at[pl.ds(0, EBLK)], S[lb],
                                  SL[lb]).wait()
            pltpu.make_async_copy(dst_h.at[pl.ds(0, EBLK)], D[db],
                                  SL[lb]).wait()
            pltpu.make_async_copy(attr_h.at[pl.ds(0, NB5)],
                                  A[lb].at[pl.ds(0, NB5)], SL[lb]).wait()

        def g_start(lb):
            pltpu.async_copy(x_h.at[S[lb]], R[lb], SG[lb])

        def g_wait(lb):
            pltpu.make_async_copy(x_h.at[S[lb]], R[lb], SG[lb]).wait()

        def scat_start(lb, db):
            pltpu.async_copy(M[lb], acc.at[D[db]], SS[lb], add=True)

        def scat_wait(lb, db):
            pltpu.make_async_copy(M[lb], acc.at[D[db]], SS[lb]).wait()

        def compute(b):
            rows = R[b]
            msg = M[b]
            attrb = A[b]

            def inner(e, carry2):
                a = attrb[pl.ds(e * 5, 16)]
                a = jnp.where(fmask, a, 0.0)
                msg[e] = jnp.maximum(rows[e] + a, 0.0)
                return carry2

            lax.fori_loop(0, EBLK, inner, 0, unroll=4)

        lin_start(0, 0, 0)
        lin_wait(0, 0)
        g_start(0)
        lin_start(1, 1, 1)

        def quad(j, carry):
            for b in range(4):
                i = 4 * j + b
                lb = b % 2
                lin_wait(1 - lb, (b + 1) % 4)
                g_start(1 - lb)
                g_wait(lb)

                @pl.when(i >= 2)
                def _():
                    scat_wait(lb, (b + 2) % 4)
                compute(lb)
                scat_start(lb, b)
                lin_start(i + 2, lb, (b + 2) % 4)
            return carry

        lax.fori_loop(0, nblk // 4, quad, 0)
        g_wait(0)
        lin_wait(1, 1)
        scat_wait(0, 2)
        scat_wait(1, 3)
        plsc.subcore_barrier()

        @pl.when(jnp.logical_and(s == 0, c == 0))
        def _():
            pltpu.sync_copy(acc.at[pl.ds(0, N)], out0)

        @pl.when(jnp.logical_and(s == 0, c == 1))
        def _():
            pltpu.sync_copy(acc.at[pl.ds(0, N)], out1)

    return k(src_p, dst_p, attr_flat, xpad, zeros_acc)


def _sc_conv2(src_p, dst_p, attr_flat, h_cm, We_cm, be_cm, zeros_acc):
    mesh = plsc.VectorSubcoreMesh(**_SC_MESH)
    per_tile = EP // 16
    nblk = per_tile // EBLK

    @functools.partial(
        pl.kernel,
        out_type=jax.ShapeDtypeStruct((8, N, 16), jnp.float32),
        mesh=mesh,
        scratch_types=[
            pltpu.VMEM((EBLK,), jnp.int32),
            pltpu.VMEM((EBLK,), jnp.int32),
            pltpu.VMEM((EBLK,), jnp.int32),
            pltpu.VMEM((EBLK,), jnp.int32),
            pltpu.VMEM((EBLK,), jnp.int32),
            pltpu.VMEM((EBLK,), jnp.int32),
            pltpu.VMEM((NB5 + 16,), jnp.float32),
            pltpu.VMEM((NB5 + 16,), jnp.float32),
            pltpu.VMEM((EBLK, 16), jnp.float32),
            pltpu.VMEM((EBLK, 16), jnp.float32),
            pltpu.VMEM((EBLK, 16), jnp.float32),
            pltpu.VMEM((EBLK, 16), jnp.float32),
            pltpu.VMEM((5, 16), jnp.float32),
            pltpu.VMEM((16,), jnp.float32),
            pltpu.VMEM_SHARED((NACC, 16), jnp.float32),
            pltpu.SemaphoreType.DMA,
            pltpu.SemaphoreType.DMA,
            pltpu.SemaphoreType.DMA,
            pltpu.SemaphoreType.DMA,
            pltpu.SemaphoreType.DMA,
            pltpu.SemaphoreType.DMA,
        ],
        compiler_params=pltpu.CompilerParams(use_tc_tiling_on_sc=False, needs_layout_passes=False),
    )
    def k(src_h, dst_h, attr_h, hcm_h, we_h, be_h, z_h, out,
          sb0, sb1, db0, db1, db2, db3, ab0, ab1, rb0, rb1, mb0, mb1,
          wev, bev, acc, sl0, sl1, sg0, sg1, ss0, ss1):
        c = lax.axis_index("c")
        s = lax.axis_index("s")
        base = s * per_tile
        S = [sb0, sb1]
        D = [db0, db1, db2, db3]
        A = [ab0, ab1]
        R = [rb0, rb1]
        M = [mb0, mb1]
        SL = [sl0, sl1]
        SG = [sg0, sg1]
        SS = [ss0, ss1]

        def lin_start(i, lb, db):
            e0 = base + jnp.minimum(i, nblk - 1) * EBLK
            pltpu.async_copy(src_h.at[pl.ds(e0, EBLK)], S[lb], SL[lb])
            pltpu.async_copy(dst_h.at[pl.ds(e0, EBLK)], D[db], SL[lb])
            pltpu.async_copy(attr_h.at[pl.ds(e0 * 5, NB5)],
                             A[lb].at[pl.ds(0, NB5)], SL[lb])

        def lin_wait(lb, db):
            pltpu.make_async_copy(src_h.at[pl.ds(0, EBLK)], S[lb],
                                  SL[lb]).wait()
            pltpu.make_async_copy(dst_h.at[pl.ds(0, EBLK)], D[db],
                                  SL[lb]).wait()
            pltpu.make_async_copy(attr_h.at[pl.ds(0, NB5)],
                                  A[lb].at[pl.ds(0, NB5)], SL[lb]).wait()

        def g_start(lb, off):
            pltpu.async_copy(hcm_h.at[pl.ds(off, N)].at[S[lb]], R[lb],
                             SG[lb])

        def g_wait(lb):
            pltpu.make_async_copy(hcm_h.at[pl.ds(0, N)].at[S[lb]], R[lb],
                                  SG[lb]).wait()

        def scat_start(lb, db):
            pltpu.async_copy(M[lb], acc.at[D[db]], SS[lb], add=True)

        def scat_wait(lb, db):
            pltpu.make_async_copy(M[lb], acc.at[D[db]], SS[lb]).wait()

        for p in range(4):
            @pl.when(s == 0)
            def _():
                pltpu.sync_copy(z_h, acc)

            @pl.when(c == 0)
            def _():
                pltpu.sync_copy(we_h.at[2 * p], wev)
                pltpu.sync_copy(be_h.at[2 * p], bev)

            @pl.when(c == 1)
            def _():
                pltpu.sync_copy(we_h.at[2 * p + 1], wev)
                pltpu.sync_copy(be_h.at[2 * p + 1], bev)
            plsc.subcore_barrier()

            off = (2 * p + c) * N
            we0 = wev[0]
            we1 = wev[1]
            we2 = wev[2]
            we3 = wev[3]
            we4 = wev[4]
            bevv = bev[pl.ds(0, 16)]
            c5 = jnp.full((16,), 5, jnp.int32)
            iv_init = tuple(
                jnp.full((16,), kk, jnp.int32) for kk in range(5))

            def compute(b):
                rows = R[b]
                msg = M[b]
                attrb = A[b]

                @plsc.parallel_loop(0, EBLK, unroll=4, carry=iv_init)
                def _loop(e, carry2):
                    j0, j1, j2, j3, j4 = carry2
                    a0 = plsc.load_gather(attrb, [j0])
                    a1 = plsc.load_gather(attrb, [j1])
                    a2 = plsc.load_gather(attrb, [j2])
                    a3 = plsc.load_gather(attrb, [j3])
                    a4 = plsc.load_gather(attrb, [j4])
                    v = rows[e] + bevv + a0 * we0 + a1 * we1 \
                        + a2 * we2 + a3 * we3 + a4 * we4
                    msg[e] = jnp.maximum(v, 0.0)
                    return (j0 + c5, j1 + c5, j2 + c5, j3 + c5, j4 + c5)

            lin_start(0, 0, 0)
            lin_wait(0, 0)
            g_start(0, off)
            lin_start(1, 1, 1)

            def quad(j, carry):
                for b in range(4):
                    i = 4 * j + b
                    lb = b % 2
                    lin_wait(1 - lb, (b + 1) % 4)
                    g_start(1 - lb, off)
                    g_wait(lb)

                    @pl.when(i >= 2)
                    def _():
                        scat_wait(lb, (b + 2) % 4)
                    compute(lb)
                    scat_start(lb, b)
                    lin_start(i + 2, lb, (b + 2) % 4)
                return carry

            lax.fori_loop(0, nblk // 4, quad, 0)
            g_wait(0)
            lin_wait(1, 1)
            scat_wait(0, 2)
            scat_wait(1, 3)
            plsc.subcore_barrier()

            @pl.when(jnp.logical_and(s == 0, c == 0))
            def _():
                pltpu.sync_copy(acc.at[pl.ds(0, N)], out.at[2 * p])

            @pl.when(jnp.logical_and(s == 0, c == 1))
            def _():
                pltpu.sync_copy(acc.at[pl.ds(0, N)], out.at[2 * p + 1])
            plsc.subcore_barrier()

    return k(src_p, dst_p, attr_flat, h_cm, We_cm, be_cm, zeros_acc)


def _mlp1(xpad, a0, a1, W1p, b1r, W2, b2r):
    def body(x_ref, a0_ref, a1_ref, w1_ref, b1_ref, w2_ref, b2_ref, o_ref):
        X = x_ref[...] + a0_ref[...] + a1_ref[...]
        t = jnp.maximum(jnp.dot(X, w1_ref[...],
                                preferred_element_type=jnp.float32)
                        + b1_ref[...], 0.0)
        o_ref[...] = jnp.maximum(jnp.dot(t, w2_ref[...],
                                         preferred_element_type=jnp.float32)
                                 + b2_ref[...], 0.0)

    return pl.pallas_call(
        body,
        grid=(NGRID,),
        in_specs=[
            pl.BlockSpec((NBLK, 16), lambda i: (i, 0)),
            pl.BlockSpec((NBLK, 16), lambda i: (i, 0)),
            pl.BlockSpec((NBLK, 16), lambda i: (i, 0)),
            pl.BlockSpec((16, 128), lambda i: (0, 0)),
            pl.BlockSpec((1, 128), lambda i: (0, 0)),
            pl.BlockSpec((128, 128), lambda i: (0, 0)),
            pl.BlockSpec((1, 128), lambda i: (0, 0)),
        ],
        out_specs=pl.BlockSpec((NBLK, 128), lambda i: (i, 0)),
        out_shape=jax.ShapeDtypeStruct((N, 128), jnp.float32),
    )(xpad, a0, a1, W1p, b1r, W2, b2r)


def _mlp2pool(h, agg2, batch3, V1, c1r, V2, c2r):
    def body(h_ref, a_ref, b_ref, v1_ref, c1_ref, v2_ref, c2_ref,
             sums_ref, cnts_ref):
        @pl.when(pl.program_id(0) == 0)
        def _():
            sums_ref[...] = jnp.zeros_like(sums_ref)
            cnts_ref[...] = jnp.zeros_like(cnts_ref)

        X = h_ref[...] + a_ref[...]
        t = jnp.maximum(jnp.dot(X, v1_ref[...],
                                preferred_element_type=jnp.float32)
                        + c1_ref[...], 0.0)
        h2 = jnp.maximum(jnp.dot(t, v2_ref[...],
                                 preferred_element_type=jnp.float32)
                         + c2_ref[...], 0.0)
        bb = b_ref[...].reshape(1, NBLK)
        ohT = (jnp.broadcast_to(bb, (B, NBLK))
               == lax.broadcasted_iota(jnp.int32, (B, NBLK), 0)
               ).astype(jnp.float32)
        sums_ref[...] += jnp.dot(ohT, h2, preferred_element_type=jnp.float32)
        cnt = jnp.sum(ohT, axis=1, keepdims=True)
        cnts_ref[...] += jnp.broadcast_to(cnt, (B, 128))

    return pl.pallas_call(
        body,
        grid=(NGRID,),
        in_specs=[
            pl.BlockSpec((NBLK, 128), lambda i: (i, 0)),
            pl.BlockSpec((NBLK, 128), lambda i: (i, 0)),
            pl.BlockSpec((1, 1, NBLK), lambda i: (i, 0, 0)),
            pl.BlockSpec((128, 128), lambda i: (0, 0)),
            pl.BlockSpec((1, 128), lambda i: (0, 0)),
            pl.BlockSpec((128, 128), lambda i: (0, 0)),
            pl.BlockSpec((1, 128), lambda i: (0, 0)),
        ],
        out_specs=[
            pl.BlockSpec((B, 128), lambda i: (0, 0)),
            pl.BlockSpec((B, 128), lambda i: (0, 0)),
        ],
        out_shape=[
            jax.ShapeDtypeStruct((B, 128), jnp.float32),
            jax.ShapeDtypeStruct((B, 128), jnp.float32),
        ],
        compiler_params=pltpu.CompilerParams(
            dimension_semantics=("arbitrary",)),
    )(h, agg2, batch3, V1, c1r, V2, c2r)


def _final(sums, cnts, recipe, len_col, stats_p, baseline, emb_pad,
           Wi, Wh, bLr, S1p, sb1r, S2, sb2r,
           H1g, H1r, H1s, H1b, hb1r, H2, hb2r, H3p, hb3r):
    def body(sums_ref, cnts_ref, rec_ref, len_ref, st_ref, base_ref, emb_ref,
             wi_ref, wh_ref, bl_ref, s1_ref, sb1_ref, s2_ref, sb2_ref,
             h1g_ref, h1r_ref, h1s_ref, h1b_ref, hb1_ref, h2_ref, hb2_ref,
             h3_ref, hb3_ref, o_ref):
        g = sums_ref[...] / jnp.maximum(cnts_ref[...], 1.0)

        idxc = jnp.clip(len_ref[...] - 1, 0, L - 1)
        wi = wi_ref[...]
        wh = wh_ref[...]
        bl = bl_ref[...]
        emb = emb_ref[...]

        def sigmoid(v):
            return 1.0 / (1.0 + jnp.exp(-v))

        rec = rec_ref[...]
        zero = jnp.zeros((B, 64), jnp.float32)
        hh, cc, sel = zero, zero, zero
        for t in range(L):
            rt = rec[:, t:t + 1]
            oh = (jnp.broadcast_to(rt, (B, VOCABP))
                  == lax.broadcasted_iota(jnp.int32, (B, VOCABP), 1)
                  ).astype(jnp.float32)
            xt = jnp.dot(oh, emb, preferred_element_type=jnp.float32)
            z = (jnp.dot(xt, wi, preferred_element_type=jnp.float32)
                 + jnp.dot(hh, wh, preferred_element_type=jnp.float32) + bl)
            i_ = sigmoid(z[:, 0:64])
            f_ = sigmoid(z[:, 64:128])
            g_ = jnp.tanh(z[:, 128:192])
            o_ = sigmoid(z[:, 192:256])
            cc = f_ * cc + i_ * g_
            hh = o_ * jnp.tanh(cc)
            sel = jnp.where(idxc == t, hh, sel)
        r = sel

        st = jnp.maximum(jnp.dot(st_ref[...], s1_ref[...],
                                 preferred_element_type=jnp.float32)
                         + sb1_ref[...], 0.0)
        s = jnp.dot(st, s2_ref[...], preferred_element_type=jnp.float32) \
            + sb2_ref[...]

        z1 = (jnp.dot(g, h1g_ref[...], preferred_element_type=jnp.float32)
              + jnp.dot(r, h1r_ref[...], preferred_element_type=jnp.float32)
              + jnp.dot(s, h1s_ref[...], preferred_element_type=jnp.float32)
              + base_ref[...] * h1b_ref[...]
              + hb1_ref[...])
        o1 = jnp.maximum(z1, 0.0)
        o2 = jnp.maximum(jnp.dot(o1, h2_ref[...],
                                 preferred_element_type=jnp.float32)
                         + hb2_ref[...], 0.0)
        o_ref[...] = jnp.dot(o2, h3_ref[...],
                             preferred_element_type=jnp.float32) + hb3_ref[...]

    return pl.pallas_call(
        body,
        out_shape=jax.ShapeDtypeStruct((B, 128), jnp.float32),
    )(sums, cnts, recipe, len_col, stats_p, baseline, emb_pad,
      Wi, Wh, bLr, S1p, sb1r, S2, sb2r,
      H1g, H1r, H1s, H1b, hb1r, H2, hb2r, H3p, hb3r)


def kernel(x, edge_index, edge_attr, batch, recipe, lengths, stats, baseline,
           W1, b1, W2, b2, We, be, V1, c1, V2, c2, emb, Wi, Wh, bL,
           S1, sb1, S2, sb2, H1, hb1, H2, hb2, H3, hb3):
    pad = EP - E
    src_p = jnp.concatenate([edge_index[0], jnp.zeros((pad,), jnp.int32)])
    dst_p = jnp.concatenate([edge_index[1],
                             jnp.full((pad,), N, jnp.int32)])
    attr_flat = jnp.concatenate([edge_attr.reshape(-1),
                                 jnp.zeros((pad * 5 + 16,), jnp.float32)])
    xpad = jnp.pad(x, ((0, 0), (0, 11)))
    zeros_acc = jnp.zeros((NACC, 16), jnp.float32)

    a0, a1 = _sc_conv1(src_p, dst_p, attr_flat, xpad, zeros_acc)

    W1p = jnp.pad(W1, ((0, 11), (0, 0)))
    h = _mlp1(xpad, a0, a1, W1p, b1.reshape(1, 128), W2, b2.reshape(1, 128))

    h_cm = h.reshape(N, 8, 16).transpose(1, 0, 2).reshape(8 * N, 16)
    We_cm = We.reshape(5, 8, 16).transpose(1, 0, 2)
    be_cm = be.reshape(8, 16)
    agg2_cm = _sc_conv2(src_p, dst_p, attr_flat, h_cm, We_cm, be_cm,
                        zeros_acc)
    agg2 = agg2_cm.transpose(1, 0, 2).reshape(N, 128)

    batch3 = batch.reshape(NGRID, 1, NBLK)
    sums, cnts = _mlp2pool(h, agg2, batch3, V1, c1.reshape(1, 128),
                           V2, c2.reshape(1, 128))

    emb_pad = jnp.pad(emb, ((0, VOCABP - (emb.shape[0])), (0, 0)))
    stats_p = jnp.pad(stats, ((0, 0), (0, 2)))
    S1p = jnp.pad(S1, ((0, 2), (0, 0)))
    H1g = H1[0:128]
    H1r = H1[128:192]
    H1s = H1[192:224]
    H1b = H1[224:225]
    H3p = jnp.pad(H3, ((0, 0), (0, 127)))
    hb3r = jnp.pad(hb3, (0, 127)).reshape(1, 128)

    out128 = _final(sums, cnts, recipe, lengths.reshape(B, 1),
                    stats_p, baseline, emb_pad,
                    Wi, Wh, bL.reshape(1, 256),
                    S1p, sb1.reshape(1, 32), S2, sb2.reshape(1, 32),
                    H1g, H1r, H1s, H1b, hb1.reshape(1, 128),
                    H2, hb2.reshape(1, 64), H3p, hb3r)
    return out128[:, 0:1]


# speedup vs baseline: 4.3119x; 1.0428x over previous
import functools

import jax
import jax.numpy as jnp
from jax import lax
from jax.experimental import pallas as pl
from jax.experimental.pallas import tpu as pltpu
from jax.experimental.pallas import tpu_sc as plsc

N = 100000
E = 3200000
EBLK = 128
EP = 784 * 32 * EBLK
NACC = 100008
B = 64
L = 50
VOCABP = 1024
NBLK = 1000
NGRID = N // NBLK

_SC_MESH = dict(core_axis_name="c", subcore_axis_name="s")


NB5 = EBLK * 5


def _sc_conv1(src_p, dst_p, attr_flat, xpad, zeros_acc):
    mesh = plsc.VectorSubcoreMesh(**_SC_MESH)
    per_tile = EP // 32
    nblk = per_tile // EBLK

    @functools.partial(
        pl.kernel,
        out_type=(jax.ShapeDtypeStruct((N, 16), jnp.float32),
                  jax.ShapeDtypeStruct((N, 16), jnp.float32)),
        mesh=mesh,
        scratch_types=[
            pltpu.VMEM((EBLK,), jnp.int32),
            pltpu.VMEM((EBLK,), jnp.int32),
            pltpu.VMEM((EBLK,), jnp.int32),
            pltpu.VMEM((EBLK,), jnp.int32),
            pltpu.VMEM((EBLK,), jnp.int32),
            pltpu.VMEM((EBLK,), jnp.int32),
            pltpu.VMEM((NB5 + 16,), jnp.float32),
            pltpu.VMEM((NB5 + 16,), jnp.float32),
            pltpu.VMEM((EBLK, 16), jnp.float32),
            pltpu.VMEM((EBLK, 16), jnp.float32),
            pltpu.VMEM((EBLK, 16), jnp.float32),
            pltpu.VMEM((EBLK, 16), jnp.float32),
            pltpu.VMEM_SHARED((NACC, 16), jnp.float32),
            pltpu.SemaphoreType.DMA,
            pltpu.SemaphoreType.DMA,
            pltpu.SemaphoreType.DMA,
            pltpu.SemaphoreType.DMA,
            pltpu.SemaphoreType.DMA,
            pltpu.SemaphoreType.DMA,
        ],
        compiler_params=pltpu.CompilerParams(use_tc_tiling_on_sc=False, needs_layout_passes=False),
    )
    def k(src_h, dst_h, attr_h, x_h, z_h, out0, out1,
          sb0, sb1, db0, db1, db2, db3, ab0, ab1, rb0, rb1, mb0, mb1, acc,
          sl0, sl1, sg0, sg1, ss0, ss1):
        c = lax.axis_index("c")
        s = lax.axis_index("s")
        S = [sb0, sb1]
        D = [db0, db1, db2, db3]
        A = [ab0, ab1]
        R = [rb0, rb1]
        M = [mb0, mb1]
        SL = [sl0, sl1]
        SG = [sg0, sg1]
        SS = [ss0, ss1]

        @pl.when(s == 0)
        def _():
            pltpu.sync_copy(z_h, acc)
        plsc.subcore_barrier()

        base = (c * 16 + s) * per_tile
        fmask = lax.broadcasted_iota(jnp.int32, (16,), 0) < 5

        def lin_start(i, lb, db):
            e0 = base + jnp.minimum(i, nblk - 1) * EBLK
            pltpu.async_copy(src_h.at[pl.ds(e0, EBLK)], S[lb], SL[lb])
            pltpu.async_copy(dst_h.at[pl.ds(e0, EBLK)], D[db], SL[lb])
            pltpu.async_copy(attr_h.at[pl.ds(e0 * 5, NB5)],
                             A[lb].at[pl.ds(0, NB5)], SL[lb])

        def lin_wait(lb, db):
            pltpu.make_async_copy(src_h.at[pl.ds(0, EBLK)], S[lb],
                                  SL[lb]).wait()
            pltpu.make_async_copy(dst_h.at[pl.ds(0, EBLK)], D[db],
                                  SL[lb]).wait()
            pltpu.make_async_copy(attr_h.at[pl.ds(0, NB5)],
                                  A[lb].at[pl.ds(0, NB5)], SL[lb]).wait()

        def g_start(lb):
            pltpu.async_copy(x_h.at[S[lb]], R[lb], SG[lb])

        def g_wait(lb):
            pltpu.make_async_copy(x_h.at[S[lb]], R[lb], SG[lb]).wait()

        def scat_start(lb, db):
            pltpu.async_copy(M[lb], acc.at[D[db]], SS[lb], add=True)

        def scat_wait(lb, db):
            pltpu.make_async_copy(M[lb], acc.at[D[db]], SS[lb]).wait()

        def compute(b):
            rows = R[b]
            msg = M[b]
            attrb = A[b]

            @plsc.parallel_loop(0, EBLK, unroll=4)
            def _loop(e):
                a = attrb[pl.ds(e * 5, 16)]
                a = jnp.where(fmask, a, 0.0)
                msg[e] = jnp.maximum(rows[e] + a, 0.0)

        lin_start(0, 0, 0)
        lin_wait(0, 0)
        g_start(0)
        lin_start(1, 1, 1)

        def quad(j, carry):
            for b in range(4):
                i = 4 * j + b
                lb = b % 2
                lin_wait(1 - lb, (b + 1) % 4)
                g_start(1 - lb)
                g_wait(lb)

                @pl.when(i >= 2)
                def _():
                    scat_wait(lb, (b + 2) % 4)
                compute(lb)
                scat_start(lb, b)
                lin_start(i + 2, lb, (b + 2) % 4)
            return carry

        lax.fori_loop(0, nblk // 4, quad, 0)
        g_wait(0)
        lin_wait(1, 1)
        scat_wait(0, 2)
        scat_wait(1, 3)
        plsc.subcore_barrier()

        @pl.when(jnp.logical_and(s == 0, c == 0))
        def _():
            pltpu.sync_copy(acc.at[pl.ds(0, N)], out0)

        @pl.when(jnp.logical_and(s == 0, c == 1))
        def _():
            pltpu.sync_copy(acc.at[pl.ds(0, N)], out1)

    return k(src_p, dst_p, attr_flat, xpad, zeros_acc)


def _sc_conv2(src_p, dst_p, attr_flat, h_cm, We_cm, be_cm, zeros_acc):
    mesh = plsc.VectorSubcoreMesh(**_SC_MESH)
    per_tile = EP // 16
    nblk = per_tile // EBLK

    @functools.partial(
        pl.kernel,
        out_type=jax.ShapeDtypeStruct((8, N, 16), jnp.float32),
        mesh=mesh,
        scratch_types=[
            pltpu.VMEM((EBLK,), jnp.int32),
            pltpu.VMEM((EBLK,), jnp.int32),
            pltpu.VMEM((EBLK,), jnp.int32),
            pltpu.VMEM((EBLK,), jnp.int32),
            pltpu.VMEM((EBLK,), jnp.int32),
            pltpu.VMEM((EBLK,), jnp.int32),
            pltpu.VMEM((NB5 + 16,), jnp.float32),
            pltpu.VMEM((NB5 + 16,), jnp.float32),
            pltpu.VMEM((EBLK, 16), jnp.float32),
            pltpu.VMEM((EBLK, 16), jnp.float32),
            pltpu.VMEM((EBLK, 16), jnp.float32),
            pltpu.VMEM((EBLK, 16), jnp.float32),
            pltpu.VMEM((5, 16), jnp.float32),
            pltpu.VMEM((16,), jnp.float32),
            pltpu.VMEM_SHARED((NACC, 16), jnp.float32),
            pltpu.SemaphoreType.DMA,
            pltpu.SemaphoreType.DMA,
            pltpu.SemaphoreType.DMA,
            pltpu.SemaphoreType.DMA,
            pltpu.SemaphoreType.DMA,
            pltpu.SemaphoreType.DMA,
        ],
        compiler_params=pltpu.CompilerParams(use_tc_tiling_on_sc=False, needs_layout_passes=False),
    )
    def k(src_h, dst_h, attr_h, hcm_h, we_h, be_h, z_h, out,
          sb0, sb1, db0, db1, db2, db3, ab0, ab1, rb0, rb1, mb0, mb1,
          wev, bev, acc, sl0, sl1, sg0, sg1, ss0, ss1):
        c = lax.axis_index("c")
        s = lax.axis_index("s")
        base = s * per_tile
        S = [sb0, sb1]
        D = [db0, db1, db2, db3]
        A = [ab0, ab1]
        R = [rb0, rb1]
        M = [mb0, mb1]
        SL = [sl0, sl1]
        SG = [sg0, sg1]
        SS = [ss0, ss1]

        def lin_start(i, lb, db):
            e0 = base + jnp.minimum(i, nblk - 1) * EBLK
            pltpu.async_copy(src_h.at[pl.ds(e0, EBLK)], S[lb], SL[lb])
            pltpu.async_copy(dst_h.at[pl.ds(e0, EBLK)], D[db], SL[lb])
            pltpu.async_copy(attr_h.at[pl.ds(e0 * 5, NB5)],
                             A[lb].at[pl.ds(0, NB5)], SL[lb])

        def lin_wait(lb, db):
            pltpu.make_async_copy(src_h.at[pl.ds(0, EBLK)], S[lb],
                                  SL[lb]).wait()
            pltpu.make_async_copy(dst_h.at[pl.ds(0, EBLK)], D[db],
                                  SL[lb]).wait()
            pltpu.make_async_copy(attr_h.at[pl.ds(0, NB5)],
                                  A[lb].at[pl.ds(0, NB5)], SL[lb]).wait()

        def g_start(lb, off):
            pltpu.async_copy(hcm_h.at[pl.ds(off, N)].at[S[lb]], R[lb],
                             SG[lb])

        def g_wait(lb):
            pltpu.make_async_copy(hcm_h.at[pl.ds(0, N)].at[S[lb]], R[lb],
                                  SG[lb]).wait()

        def scat_start(lb, db):
            pltpu.async_copy(M[lb], acc.at[D[db]], SS[lb], add=True)

        def scat_wait(lb, db):
            pltpu.make_async_copy(M[lb], acc.at[D[db]], SS[lb]).wait()

        for p in range(4):
            @pl.when(s == 0)
            def _():
                pltpu.sync_copy(z_h, acc)

            @pl.when(c == 0)
            def _():
                pltpu.sync_copy(we_h.at[2 * p], wev)
                pltpu.sync_copy(be_h.at[2 * p], bev)

            @pl.when(c == 1)
            def _():
                pltpu.sync_copy(we_h.at[2 * p + 1], wev)
                pltpu.sync_copy(be_h.at[2 * p + 1], bev)
            plsc.subcore_barrier()

            off = (2 * p + c) * N
            we0 = wev[0]
            we1 = wev[1]
            we2 = wev[2]
            we3 = wev[3]
            we4 = wev[4]
            bevv = bev[pl.ds(0, 16)]
            c5 = jnp.full((16,), 5, jnp.int32)
            iv_init = tuple(
                jnp.full((16,), kk, jnp.int32) for kk in range(5))

            def compute(b):
                rows = R[b]
                msg = M[b]
                attrb = A[b]

                @plsc.parallel_loop(0, EBLK, unroll=4, carry=iv_init)
                def _loop(e, carry2):
                    j0, j1, j2, j3, j4 = carry2
                    a0 = plsc.load_gather(attrb, [j0])
                    a1 = plsc.load_gather(attrb, [j1])
                    a2 = plsc.load_gather(attrb, [j2])
                    a3 = plsc.load_gather(attrb, [j3])
                    a4 = plsc.load_gather(attrb, [j4])
                    v = rows[e] + bevv + a0 * we0 + a1 * we1 \
                        + a2 * we2 + a3 * we3 + a4 * we4
                    msg[e] = jnp.maximum(v, 0.0)
                    return (j0 + c5, j1 + c5, j2 + c5, j3 + c5, j4 + c5)

            lin_start(0, 0, 0)
            lin_wait(0, 0)
            g_start(0, off)
            lin_start(1, 1, 1)

            def quad(j, carry):
                for b in range(4):
                    i = 4 * j + b
                    lb = b % 2
                    lin_wait(1 - lb, (b + 1) % 4)
                    g_start(1 - lb, off)
                    g_wait(lb)

                    @pl.when(i >= 2)
                    def _():
                        scat_wait(lb, (b + 2) % 4)
                    compute(lb)
                    scat_start(lb, b)
                    lin_start(i + 2, lb, (b + 2) % 4)
                return carry

            lax.fori_loop(0, nblk // 4, quad, 0)
            g_wait(0)
            lin_wait(1, 1)
            scat_wait(0, 2)
            scat_wait(1, 3)
            plsc.subcore_barrier()

            @pl.when(jnp.logical_and(s == 0, c == 0))
            def _():
                pltpu.sync_copy(acc.at[pl.ds(0, N)], out.at[2 * p])

            @pl.when(jnp.logical_and(s == 0, c == 1))
            def _():
                pltpu.sync_copy(acc.at[pl.ds(0, N)], out.at[2 * p + 1])
            plsc.subcore_barrier()

    return k(src_p, dst_p, attr_flat, h_cm, We_cm, be_cm, zeros_acc)


def _mlp1(xpad, a0, a1, W1p, b1r, W2, b2r):
    def body(x_ref, a0_ref, a1_ref, w1_ref, b1_ref, w2_ref, b2_ref, o_ref):
        X = x_ref[...] + a0_ref[...] + a1_ref[...]
        t = jnp.maximum(jnp.dot(X, w1_ref[...],
                                preferred_element_type=jnp.float32)
                        + b1_ref[...], 0.0)
        o_ref[...] = jnp.maximum(jnp.dot(t, w2_ref[...],
                                         preferred_element_type=jnp.float32)
                                 + b2_ref[...], 0.0)

    return pl.pallas_call(
        body,
        grid=(NGRID,),
        in_specs=[
            pl.BlockSpec((NBLK, 16), lambda i: (i, 0)),
            pl.BlockSpec((NBLK, 16), lambda i: (i, 0)),
            pl.BlockSpec((NBLK, 16), lambda i: (i, 0)),
            pl.BlockSpec((16, 128), lambda i: (0, 0)),
            pl.BlockSpec((1, 128), lambda i: (0, 0)),
            pl.BlockSpec((128, 128), lambda i: (0, 0)),
            pl.BlockSpec((1, 128), lambda i: (0, 0)),
        ],
        out_specs=pl.BlockSpec((NBLK, 128), lambda i: (i, 0)),
        out_shape=jax.ShapeDtypeStruct((N, 128), jnp.float32),
    )(xpad, a0, a1, W1p, b1r, W2, b2r)


def _mlp2pool(h, agg2, batch3, V1, c1r, V2, c2r):
    def body(h_ref, a_ref, b_ref, v1_ref, c1_ref, v2_ref, c2_ref,
             sums_ref, cnts_ref):
        @pl.when(pl.program_id(0) == 0)
        def _():
            sums_ref[...] = jnp.zeros_like(sums_ref)
            cnts_ref[...] = jnp.zeros_like(cnts_ref)

        X = h_ref[...] + a_ref[...]
        t = jnp.maximum(jnp.dot(X, v1_ref[...],
                                preferred_element_type=jnp.float32)
                        + c1_ref[...], 0.0)
        h2 = jnp.maximum(jnp.dot(t, v2_ref[...],
                                 preferred_element_type=jnp.float32)
                         + c2_ref[...], 0.0)
        bb = b_ref[...].reshape(1, NBLK)
        ohT = (jnp.broadcast_to(bb, (B, NBLK))
               == lax.broadcasted_iota(jnp.int32, (B, NBLK), 0)
               ).astype(jnp.float32)
        sums_ref[...] += jnp.dot(ohT, h2, preferred_element_type=jnp.float32)
        cnt = jnp.sum(ohT, axis=1, keepdims=True)
        cnts_ref[...] += jnp.broadcast_to(cnt, (B, 128))

    return pl.pallas_call(
        body,
        grid=(NGRID,),
        in_specs=[
            pl.BlockSpec((NBLK, 128), lambda i: (i, 0)),
            pl.BlockSpec((NBLK, 128), lambda i: (i, 0)),
            pl.BlockSpec((1, 1, NBLK), lambda i: (i, 0, 0)),
            pl.BlockSpec((128, 128), lambda i: (0, 0)),
            pl.BlockSpec((1, 128), lambda i: (0, 0)),
            pl.BlockSpec((128, 128), lambda i: (0, 0)),
            pl.BlockSpec((1, 128), lambda i: (0, 0)),
        ],
        out_specs=[
            pl.BlockSpec((B, 128), lambda i: (0, 0)),
            pl.BlockSpec((B, 128), lambda i: (0, 0)),
        ],
        out_shape=[
            jax.ShapeDtypeStruct((B, 128), jnp.float32),
            jax.ShapeDtypeStruct((B, 128), jnp.float32),
        ],
        compiler_params=pltpu.CompilerParams(
            dimension_semantics=("arbitrary",)),
    )(h, agg2, batch3, V1, c1r, V2, c2r)


def _final(sums, cnts, recipe, len_col, stats_p, baseline, emb_pad,
           Wi, Wh, bLr, S1p, sb1r, S2, sb2r,
           H1g, H1r, H1s, H1b, hb1r, H2, hb2r, H3p, hb3r):
    def body(sums_ref, cnts_ref, rec_ref, len_ref, st_ref, base_ref, emb_ref,
             wi_ref, wh_ref, bl_ref, s1_ref, sb1_ref, s2_ref, sb2_ref,
             h1g_ref, h1r_ref, h1s_ref, h1b_ref, hb1_ref, h2_ref, hb2_ref,
             h3_ref, hb3_ref, o_ref):
        g = sums_ref[...] / jnp.maximum(cnts_ref[...], 1.0)

        idxc = jnp.clip(len_ref[...] - 1, 0, L - 1)
        wi = wi_ref[...]
        wh = wh_ref[...]
        bl = bl_ref[...]
        emb = emb_ref[...]

        def sigmoid(v):
            return 1.0 / (1.0 + jnp.exp(-v))

        rec = rec_ref[...]
        zero = jnp.zeros((B, 64), jnp.float32)
        hh, cc, sel = zero, zero, zero
        for t in range(L):
            rt = rec[:, t:t + 1]
            oh = (jnp.broadcast_to(rt, (B, VOCABP))
                  == lax.broadcasted_iota(jnp.int32, (B, VOCABP), 1)
                  ).astype(jnp.float32)
            xt = jnp.dot(oh, emb, preferred_element_type=jnp.float32)
            z = (jnp.dot(xt, wi, preferred_element_type=jnp.float32)
                 + jnp.dot(hh, wh, preferred_element_type=jnp.float32) + bl)
            i_ = sigmoid(z[:, 0:64])
            f_ = sigmoid(z[:, 64:128])
            g_ = jnp.tanh(z[:, 128:192])
            o_ = sigmoid(z[:, 192:256])
            cc = f_ * cc + i_ * g_
            hh = o_ * jnp.tanh(cc)
            sel = jnp.where(idxc == t, hh, sel)
        r = sel

        st = jnp.maximum(jnp.dot(st_ref[...], s1_ref[...],
                                 preferred_element_type=jnp.float32)
                         + sb1_ref[...], 0.0)
        s = jnp.dot(st, s2_ref[...], preferred_element_type=jnp.float32) \
            + sb2_ref[...]

        z1 = (jnp.dot(g, h1g_ref[...], preferred_element_type=jnp.float32)
              + jnp.dot(r, h1r_ref[...], preferred_element_type=jnp.float32)
              + jnp.dot(s, h1s_ref[...], preferred_element_type=jnp.float32)
              + base_ref[...] * h1b_ref[...]
              + hb1_ref[...])
        o1 = jnp.maximum(z1, 0.0)
        o2 = jnp.maximum(jnp.dot(o1, h2_ref[...],
                                 preferred_element_type=jnp.float32)
                         + hb2_ref[...], 0.0)
        o_ref[...] = jnp.dot(o2, h3_ref[...],
                             preferred_element_type=jnp.float32) + hb3_ref[...]

    return pl.pallas_call(
        body,
        out_shape=jax.ShapeDtypeStruct((B, 128), jnp.float32),
    )(sums, cnts, recipe, len_col, stats_p, baseline, emb_pad,
      Wi, Wh, bLr, S1p, sb1r, S2, sb2r,
      H1g, H1r, H1s, H1b, hb1r, H2, hb2r, H3p, hb3r)


def kernel(x, edge_index, edge_attr, batch, recipe, lengths, stats, baseline,
           W1, b1, W2, b2, We, be, V1, c1, V2, c2, emb, Wi, Wh, bL,
           S1, sb1, S2, sb2, H1, hb1, H2, hb2, H3, hb3):
    pad = EP - E
    src_p = jnp.concatenate([edge_index[0], jnp.zeros((pad,), jnp.int32)])
    dst_p = jnp.concatenate([edge_index[1],
                             jnp.full((pad,), N, jnp.int32)])
    attr_flat = jnp.concatenate([edge_attr.reshape(-1),
                                 jnp.zeros((pad * 5 + 16,), jnp.float32)])
    xpad = jnp.pad(x, ((0, 0), (0, 11)))
    zeros_acc = jnp.zeros((NACC, 16), jnp.float32)

    a0, a1 = _sc_conv1(src_p, dst_p, attr_flat, xpad, zeros_acc)

    W1p = jnp.pad(W1, ((0, 11), (0, 0)))
    h = _mlp1(xpad, a0, a1, W1p, b1.reshape(1, 128), W2, b2.reshape(1, 128))

    h_cm = h.reshape(N, 8, 16).transpose(1, 0, 2).reshape(8 * N, 16)
    We_cm = We.reshape(5, 8, 16).transpose(1, 0, 2)
    be_cm = be.reshape(8, 16)
    agg2_cm = _sc_conv2(src_p, dst_p, attr_flat, h_cm, We_cm, be_cm,
                        zeros_acc)
    agg2 = agg2_cm.transpose(1, 0, 2).reshape(N, 128)

    batch3 = batch.reshape(NGRID, 1, NBLK)
    sums, cnts = _mlp2pool(h, agg2, batch3, V1, c1.reshape(1, 128),
                           V2, c2.reshape(1, 128))

    emb_pad = jnp.pad(emb, ((0, VOCABP - (emb.shape[0])), (0, 0)))
    stats_p = jnp.pad(stats, ((0, 0), (0, 2)))
    S1p = jnp.pad(S1, ((0, 2), (0, 0)))
    H1g = H1[0:128]
    H1r = H1[128:192]
    H1s = H1[192:224]
    H1b = H1[224:225]
    H3p = jnp.pad(H3, ((0, 0), (0, 127)))
    hb3r = jnp.pad(hb3, (0, 127)).reshape(1, 128)

    out128 = _final(sums, cnts, recipe, lengths.reshape(B, 1),
                    stats_p, baseline, emb_pad,
                    Wi, Wh, bL.reshape(1, 256),
                    S1p, sb1.reshape(1, 32), S2, sb2.reshape(1, 32),
                    H1g, H1r, H1s, H1b, hb1.reshape(1, 128),
                    H2, hb2.reshape(1, 64), H3p, hb3r)
    return out128[:, 0:1]
